# Initial kernel scaffold; baseline (speedup 1.0000x reference)
import jax
import jax.numpy as jnp
from jax.experimental import pallas as pl


def kernel(x, edge_index, edge_trust_score, edge_query_embedding, c1_nW, c1_nb, c1_e1W, c1_e1b, c1_e2W, c1_e2b, c2_nW, c2_nb, c2_e1W, c2_e1b, c2_e2W, c2_e2b, p1_W, p1_b, p2_W, p2_b):
    raise NotImplementedError("write your pallas kernel here")


# speedup vs baseline: 2.2885x; 2.2885x over previous
import functools

import jax
import jax.numpy as jnp
from jax import lax
from jax.experimental import pallas as pl
from jax.experimental.pallas import tpu as pltpu
from jax.experimental.pallas import tpu_sc as plsc

_F32 = jnp.float32

_IW = 80
_S_K = 320
_G_K = 640
_PRED_K = 320


def _edge_pre_body(trust_ref, query_ref, w1a_ref, b1a_ref, w1b_ref, b1b_ref,
                   wq_ref, p1b_ref, g1_ref, g2_ref, p_ref):
    ea = jnp.concatenate([trust_ref[...], query_ref[...]], axis=1)
    be = ea.shape[0]
    t1 = jnp.maximum(
        jnp.dot(ea, w1a_ref[...], preferred_element_type=_F32) + b1a_ref[...], 0.0)
    t2 = jnp.maximum(
        jnp.dot(ea, w1b_ref[...], preferred_element_type=_F32) + b1b_ref[...], 0.0)
    ones = jnp.ones((be, 1), _F32)
    pad = jnp.zeros((be, 13), _F32)
    g1_ref[...] = jnp.concatenate([t1, ones, pad], axis=1)
    g2_ref[...] = jnp.concatenate([t2, ones, pad], axis=1)
    p_ref[...] = (
        jnp.dot(query_ref[...], wq_ref[...], preferred_element_type=_F32)
        + p1b_ref[...])


def _conv_update_body(sp_ref, gp_ref, e2w_ref, nw_ref, nb_ref, h_ref):
    s = sp_ref[0] + sp_ref[1]
    g = gp_ref[0] + gp_ref[1]
    aggr = s + jnp.dot(g, e2w_ref[...], preferred_element_type=_F32)
    h_ref[...] = jnp.maximum(
        jnp.dot(aggr, nw_ref[...], preferred_element_type=_F32) + nb_ref[...], 0.0)


def _conv_ab_body(sp_ref, gp_ref, e2w_ref, nw_ref, nb_ref, wsrc_ref, wdst_ref,
                  a_ref, b_ref):
    s = sp_ref[0] + sp_ref[1]
    g = gp_ref[0] + gp_ref[1]
    aggr = s + jnp.dot(g, e2w_ref[...], preferred_element_type=_F32)
    h2 = jnp.maximum(
        jnp.dot(aggr, nw_ref[...], preferred_element_type=_F32) + nb_ref[...], 0.0)
    a_ref[...] = jnp.dot(h2, wsrc_ref[...], preferred_element_type=_F32)
    b_ref[...] = jnp.dot(h2, wdst_ref[...], preferred_element_type=_F32)


def _tile_ids():
    c = lax.axis_index("core")
    s = lax.axis_index("subcore")
    return c, s, c * 16 + s


def _zero_rows(buf, nrows, width16):
    zero16 = jnp.zeros((16,), _F32)

    @pl.loop(0, nrows)
    def _(i):
        for j in range(width16):
            buf[i, pl.ds(j * 16, 16)] = zero16


def _zero_shared(stage, sh, s, n):
    nt = (n // 16) // 8 * 8
    rem = n - nt * 16
    nst = nt // 3
    for i in range(3):
        pltpu.sync_copy(stage.at[pl.ds(0, nst)],
                        sh.at[pl.ds(s * nt + i * nst, nst)])
    if rem:
        @pl.when(s == 15)
        def _():
            pltpu.sync_copy(stage.at[pl.ds(0, rem)], sh.at[pl.ds(nt * 16, rem)])


def _write_shared(sh, out_hbm, c, s, n):
    nt = (n // 16) // 8 * 8
    rem = n - nt * 16
    pltpu.sync_copy(sh.at[pl.ds(s * nt, nt)], out_hbm.at[c, pl.ds(s * nt, nt)])
    if rem:
        @pl.when(s == 15)
        def _():
            pltpu.sync_copy(sh.at[pl.ds(nt * 16, rem)],
                            out_hbm.at[c, pl.ds(nt * 16, rem)])


def _sc_srows_body(n_chunks, n, x_hbm, src_hbm, dst_hbm, sp_hbm,
                   idx_s, idx_d, rows, s_sh):
    c, s, w = _tile_ids()
    ksub = _S_K // _IW

    _zero_rows(rows, ((n // 16) // 8 * 8) // 3, 8)
    _zero_shared(rows, s_sh, s, n)
    plsc.subcore_barrier()

    @pl.loop(w, n_chunks, step=32)
    def _(chunk):
        pltpu.sync_copy(src_hbm.at[chunk], idx_s)
        pltpu.sync_copy(dst_hbm.at[chunk], idx_d)
        for j in range(ksub):
            pltpu.sync_copy(x_hbm.at[idx_s.at[j]], rows.at[pl.ds(j * _IW, _IW)])
        for j in range(ksub):
            pltpu.sync_copy(rows.at[pl.ds(j * _IW, _IW)], s_sh.at[idx_d.at[j]],
                            add=True)

    plsc.subcore_barrier()
    _write_shared(s_sh, sp_hbm, c, s, n)


def _sc_gpair_body(n_chunks, n, dst_hbm, g1_hbm, g2_hbm, gp1_hbm, gp2_hbm,
                   idx_d, g1buf, g2buf, g1_sh, g2_sh):
    c, s, w = _tile_ids()
    ksub = _G_K // _IW

    _zero_rows(g1buf, ((n // 16) // 8 * 8) // 3, 3)
    _zero_shared(g1buf, g1_sh, s, n)
    _zero_shared(g1buf, g2_sh, s, n)
    plsc.subcore_barrier()

    @pl.loop(w, n_chunks, step=32)
    def _(chunk):
        base = chunk * _G_K
        pltpu.sync_copy(dst_hbm.at[chunk], idx_d)
        pltpu.sync_copy(g1_hbm.at[pl.ds(base, _G_K)], g1buf)
        pltpu.sync_copy(g2_hbm.at[pl.ds(base, _G_K)], g2buf)
        for j in range(ksub):
            pltpu.sync_copy(g1buf.at[pl.ds(j * _IW, _IW)],
                            g1_sh.at[idx_d.at[j]], add=True)
            pltpu.sync_copy(g2buf.at[pl.ds(j * _IW, _IW)],
                            g2_sh.at[idx_d.at[j]], add=True)

    plsc.subcore_barrier()
    _write_shared(g1_sh, gp1_hbm, c, s, n)
    _write_shared(g2_sh, gp2_hbm, c, s, n)


def _sc_pred_body(n_chunks, a_hbm, b_hbm, p_hbm, src_hbm, dst_hbm, w2_hbm,
                  out_hbm, idx_s, idx_d, abuf, bbuf, pbuf, obuf, tmp, w2v):
    c, s, w = _tile_ids()
    pltpu.sync_copy(w2_hbm, w2v)
    p2bv = w2v[pl.ds(128, 16)]
    lane = lax.iota(jnp.int32, 16)
    zlane = jnp.zeros((16,), jnp.int32)
    halves = (_G_K // _PRED_K, _PRED_K // _IW)

    @pl.loop(w, n_chunks, step=32)
    def _(chunk):
        pltpu.sync_copy(src_hbm.at[chunk], idx_s)
        pltpu.sync_copy(dst_hbm.at[chunk], idx_d)
        for half in range(halves[0]):
            base = chunk * _G_K + half * _PRED_K
            for j in range(halves[1]):
                jj = half * halves[1] + j
                pltpu.sync_copy(a_hbm.at[idx_s.at[jj]],
                                abuf.at[pl.ds(j * _IW, _IW)])
                pltpu.sync_copy(b_hbm.at[idx_d.at[jj]],
                                bbuf.at[pl.ds(j * _IW, _IW)])
            pltpu.sync_copy(p_hbm.at[pl.ds(base, _PRED_K)], pbuf)

            @pl.loop(0, _PRED_K)
            def _(e):
                acc = jnp.zeros((16,), _F32)
                for j in range(8):
                    sl = pl.ds(j * 16, 16)
                    u = abuf[e, sl] + bbuf[e, sl] + pbuf[e, sl]
                    u = jnp.maximum(u, 0.0)
                    acc = acc + u * w2v[sl]
                tmp[e, :] = jnp.full((16,), jnp.sum(acc), _F32)

            @pl.loop(0, _PRED_K // 16)
            def _(i):
                t = plsc.load_gather(tmp, [i * 16 + lane, zlane]) + p2bv
                obuf[pl.ds(i * 16, 16)] = 1.0 / (1.0 + jnp.exp(-t))

            pltpu.sync_copy(obuf, out_hbm.at[pl.ds(base, _PRED_K)])


def kernel(x, edge_index, edge_trust_score, edge_query_embedding,
           c1_nW, c1_nb, c1_e1W, c1_e1b, c1_e2W, c1_e2b,
           c2_nW, c2_nb, c2_e1W, c2_e1b, c2_e2W, c2_e2b,
           p1_W, p1_b, p2_W, p2_b):
    n, din = x.shape
    e = edge_index.shape[1]
    h_dim = c1_nW.shape[1]
    dout = c2_nW.shape[1]
    q = edge_query_embedding.shape[1]

    src = edge_index[0]
    dst = edge_index[1]
    src_s = src.reshape(e // _S_K, _S_K // _IW, _IW)
    dst_s = dst.reshape(e // _S_K, _S_K // _IW, _IW)
    src_g = src.reshape(e // _G_K, _G_K // _IW, _IW)
    dst_g = dst.reshape(e // _G_K, _G_K // _IW, _IW)

    def pad_e2(e2w, e2b):
        return jnp.concatenate(
            [e2w, e2b[None, :], jnp.zeros((13, e2w.shape[1]), _F32)], axis=0)

    c1_e2Wp = pad_e2(c1_e2W, c1_e2b)
    c2_e2Wp = pad_e2(c2_e2W, c2_e2b)

    be = 2000
    g_w = c1_e1W.shape[1] + 14
    edge_pre = pl.pallas_call(
        _edge_pre_body,
        grid=(e // be,),
        in_specs=[
            pl.BlockSpec((be, 1), lambda i: (i, 0)),
            pl.BlockSpec((be, q), lambda i: (i, 0)),
            pl.BlockSpec(c1_e1W.shape, lambda i: (0, 0)),
            pl.BlockSpec((1, 34), lambda i: (0, 0)),
            pl.BlockSpec(c2_e1W.shape, lambda i: (0, 0)),
            pl.BlockSpec((1, 34), lambda i: (0, 0)),
            pl.BlockSpec((q, h_dim), lambda i: (0, 0)),
            pl.BlockSpec((1, h_dim), lambda i: (0, 0)),
        ],
        out_specs=[
            pl.BlockSpec((be, g_w), lambda i: (i, 0)),
            pl.BlockSpec((be, g_w), lambda i: (i, 0)),
            pl.BlockSpec((be, h_dim), lambda i: (i, 0)),
        ],
        out_shape=[
            jax.ShapeDtypeStruct((e, g_w), _F32),
            jax.ShapeDtypeStruct((e, g_w), _F32),
            jax.ShapeDtypeStruct((e, h_dim), _F32),
        ],
    )
    g1x, g2x, p_tab = edge_pre(
        edge_trust_score, edge_query_embedding,
        c1_e1W, c1_e1b.reshape(1, -1), c2_e1W, c2_e1b.reshape(1, -1),
        p1_W[2 * dout:], p1_b.reshape(1, -1))

    mesh = plsc.VectorSubcoreMesh(core_axis_name="core",
                                  subcore_axis_name="subcore")
    sc_params = pltpu.CompilerParams(use_tc_tiling_on_sc=False,
                                     needs_layout_passes=False)

    sc_srows = pl.kernel(
        functools.partial(_sc_srows_body, e // _S_K, n),
        out_type=jax.ShapeDtypeStruct((2, n, din), _F32),
        mesh=mesh,
        scratch_types=[
            pltpu.VMEM((_S_K // _IW, _IW), jnp.int32),
            pltpu.VMEM((_S_K // _IW, _IW), jnp.int32),
            pltpu.VMEM((_S_K, din), _F32),
            pltpu.VMEM_SHARED((n, din), _F32),
        ],
        compiler_params=sc_params,
    )

    sc_gpair = pl.kernel(
        functools.partial(_sc_gpair_body, e // _G_K, n),
        out_type=(jax.ShapeDtypeStruct((2, n, g_w), _F32),
                  jax.ShapeDtypeStruct((2, n, g_w), _F32)),
        mesh=mesh,
        scratch_types=[
            pltpu.VMEM((_G_K // _IW, _IW), jnp.int32),
            pltpu.VMEM((_G_K, g_w), _F32),
            pltpu.VMEM((_G_K, g_w), _F32),
            pltpu.VMEM_SHARED((n, g_w), _F32),
            pltpu.VMEM_SHARED((n, g_w), _F32),
        ],
        compiler_params=sc_params,
    )

    bn = 2000
    conv_update = pl.pallas_call(
        _conv_update_body,
        grid=(n // bn,),
        in_specs=[
            pl.BlockSpec((2, bn, din), lambda i: (0, i, 0)),
            pl.BlockSpec((2, bn, g_w), lambda i: (0, i, 0)),
            pl.BlockSpec((g_w, din), lambda i: (0, 0)),
            pl.BlockSpec((din, h_dim), lambda i: (0, 0)),
            pl.BlockSpec((1, h_dim), lambda i: (0, 0)),
        ],
        out_specs=pl.BlockSpec((bn, h_dim), lambda i: (i, 0)),
        out_shape=jax.ShapeDtypeStruct((n, h_dim), _F32),
    )
    conv_ab = pl.pallas_call(
        _conv_ab_body,
        grid=(n // bn,),
        in_specs=[
            pl.BlockSpec((2, bn, h_dim), lambda i: (0, i, 0)),
            pl.BlockSpec((2, bn, g_w), lambda i: (0, i, 0)),
            pl.BlockSpec((g_w, h_dim), lambda i: (0, 0)),
            pl.BlockSpec((h_dim, dout), lambda i: (0, 0)),
            pl.BlockSpec((1, dout), lambda i: (0, 0)),
            pl.BlockSpec((dout, h_dim), lambda i: (0, 0)),
            pl.BlockSpec((dout, h_dim), lambda i: (0, 0)),
        ],
        out_specs=[
            pl.BlockSpec((bn, h_dim), lambda i: (i, 0)),
            pl.BlockSpec((bn, h_dim), lambda i: (i, 0)),
        ],
        out_shape=[
            jax.ShapeDtypeStruct((n, h_dim), _F32),
            jax.ShapeDtypeStruct((n, h_dim), _F32),
        ],
    )

    gp1, gp2 = sc_gpair(dst_g, g1x, g2x)
    sp1 = sc_srows(x, src_s, dst_s)
    h = conv_update(sp1, gp1, c1_e2Wp, c1_nW, c1_nb.reshape(1, -1))
    sp2 = sc_srows(h, src_s, dst_s)
    a_tab, b_tab = conv_ab(sp2, gp2, c2_e2Wp, c2_nW, c2_nb.reshape(1, -1),
                           p1_W[:dout], p1_W[dout:2 * dout])

    w2ext = jnp.concatenate(
        [p2_W[:, 0], jnp.full((16,), p2_b[0], _F32)])
    sc_pred = pl.kernel(
        functools.partial(_sc_pred_body, e // _G_K),
        out_type=jax.ShapeDtypeStruct((e,), _F32),
        mesh=mesh,
        scratch_types=[
            pltpu.VMEM((_G_K // _IW, _IW), jnp.int32),
            pltpu.VMEM((_G_K // _IW, _IW), jnp.int32),
            pltpu.VMEM((_PRED_K, h_dim), _F32),
            pltpu.VMEM((_PRED_K, h_dim), _F32),
            pltpu.VMEM((_PRED_K, h_dim), _F32),
            pltpu.VMEM((_PRED_K,), _F32),
            pltpu.VMEM((_PRED_K, 16), _F32),
            pltpu.VMEM((144,), _F32),
        ],
        compiler_params=sc_params,
    )
    out = sc_pred(a_tab, b_tab, p_tab, src_g, dst_g, w2ext)
    return out.reshape(e, 1)


# speedup vs baseline: 2.6850x; 1.1733x over previous
import functools

import jax
import jax.numpy as jnp
from jax import lax
from jax.experimental import pallas as pl
from jax.experimental.pallas import tpu as pltpu
from jax.experimental.pallas import tpu_sc as plsc

_F32 = jnp.float32

_IW = 80
_NW = 32


def _edge_pre_body(trust_ref, query_ref, w1a_ref, b1a_ref, w1b_ref, b1b_ref,
                   wq_ref, p1b_ref, g_ref, p_ref):
    ea = jnp.concatenate([trust_ref[...], query_ref[...]], axis=1)
    be = ea.shape[0]
    t1 = jnp.maximum(
        jnp.dot(ea, w1a_ref[...], preferred_element_type=_F32) + b1a_ref[...], 0.0)
    t2 = jnp.maximum(
        jnp.dot(ea, w1b_ref[...], preferred_element_type=_F32) + b1b_ref[...], 0.0)
    ones = jnp.ones((be, 1), _F32)
    pad = jnp.zeros((be, 13), _F32)
    g_ref[...] = jnp.concatenate([t1, ones, pad, t2, ones, pad], axis=1)
    p_ref[...] = (
        jnp.dot(query_ref[...], wq_ref[...], preferred_element_type=_F32)
        + p1b_ref[...])


def _conv_update_body(sp_ref, gp_ref, e2w_ref, nw_ref, nb_ref, h_ref):
    s = sp_ref[0] + sp_ref[1]
    g = gp_ref[0] + gp_ref[1]
    aggr = s + jnp.dot(g, e2w_ref[...], preferred_element_type=_F32)
    h_ref[...] = jnp.maximum(
        jnp.dot(aggr, nw_ref[...], preferred_element_type=_F32) + nb_ref[...], 0.0)


def _conv_ab_body(sp_ref, gp_ref, e2w_ref, nw_ref, nb_ref, wsrc_ref, wdst_ref,
                  a_ref, b_ref):
    s = sp_ref[0] + sp_ref[1]
    g = gp_ref[0] + gp_ref[1]
    aggr = s + jnp.dot(g, e2w_ref[...], preferred_element_type=_F32)
    h2 = jnp.maximum(
        jnp.dot(aggr, nw_ref[...], preferred_element_type=_F32) + nb_ref[...], 0.0)
    a_ref[...] = jnp.dot(h2, wsrc_ref[...], preferred_element_type=_F32)
    b_ref[...] = jnp.dot(h2, wdst_ref[...], preferred_element_type=_F32)


def _tile_ids():
    c = lax.axis_index("core")
    s = lax.axis_index("subcore")
    return c, s, c * 16 + s


def _zero_rows(buf, nrows, width16):
    zero16 = jnp.zeros((16,), _F32)

    @pl.loop(0, nrows)
    def _(i):
        for j in range(width16):
            buf[i, pl.ds(j * 16, 16)] = zero16


def _zero_shared(stage, sh, s, n):
    nt = (n // 16) // 8 * 8
    rem = n - nt * 16
    full, part = divmod(nt, _IW)

    @pl.loop(0, full)
    def _(i):
        pltpu.sync_copy(stage.at[pl.ds(0, _IW)],
                        sh.at[pl.ds(s * nt + i * _IW, _IW)])

    if part:
        pltpu.sync_copy(stage.at[pl.ds(0, part)],
                        sh.at[pl.ds(s * nt + full * _IW, part)])
    if rem:
        @pl.when(s == 15)
        def _():
            pltpu.sync_copy(stage.at[pl.ds(0, rem)], sh.at[pl.ds(nt * 16, rem)])


def _write_shared(sh, out_hbm, c, s, n):
    nt = (n // 16) // 8 * 8
    rem = n - nt * 16
    pltpu.sync_copy(sh.at[pl.ds(s * nt, nt)], out_hbm.at[c, pl.ds(s * nt, nt)])
    if rem:
        @pl.when(s == 15)
        def _():
            pltpu.sync_copy(sh.at[pl.ds(nt * 16, rem)],
                            out_hbm.at[c, pl.ds(nt * 16, rem)])


def _sc_srows_body(ept, n, x_hbm, src_hbm, dst_hbm, sp_hbm,
                   idx_s, idx_d, rows0, rows1, s_sh, gs0, gs1, ss0, ss1):
    c, s, w = _tile_ids()
    nch = ept // _IW

    _zero_rows(rows0, _IW, 8)
    _zero_shared(rows0, s_sh, s, n)
    plsc.subcore_barrier()

    r0 = w * nch

    def chunk_pair(k0, npair):
        pltpu.sync_copy(src_hbm.at[pl.ds(r0 + k0, npair)],
                        idx_s.at[pl.ds(0, npair)])
        pltpu.sync_copy(dst_hbm.at[pl.ds(r0 + k0, npair)],
                        idx_d.at[pl.ds(0, npair)])
        dg0 = pltpu.async_copy(x_hbm.at[idx_s.at[0]], rows0, gs0)
        if npair == 2:
            dg1 = pltpu.async_copy(x_hbm.at[idx_s.at[1]], rows1, gs1)
        dg0.wait()
        dsc0 = pltpu.async_copy(rows0, s_sh.at[idx_d.at[0]], ss0, add=True)
        if npair == 2:
            dg1.wait()
            dsc1 = pltpu.async_copy(rows1, s_sh.at[idx_d.at[1]], ss1, add=True)
        dsc0.wait()
        if npair == 2:
            dsc1.wait()

    @pl.loop(0, nch // 2)
    def _(i):
        chunk_pair(2 * i, 2)

    if nch % 2:
        chunk_pair(nch - 1, 1)

    plsc.subcore_barrier()
    _write_shared(s_sh, sp_hbm, c, s, n)


def _sc_gpack_body(ept, n, g_hbm, dst_hbm, gp_hbm,
                   idx_d, gbuf0, gbuf1, g_sh, gl0, gl1, gs0, gs1):
    c, s, w = _tile_ids()
    nch = ept // _IW

    _zero_rows(gbuf0, _IW, 6)
    _zero_shared(gbuf0, g_sh, s, n)
    plsc.subcore_barrier()

    e0 = w * ept

    def chunk_pair(k0, npair):
        pltpu.sync_copy(dst_hbm.at[pl.ds(w * nch + k0, npair)],
                        idx_d.at[pl.ds(0, npair)])
        dl0 = pltpu.async_copy(g_hbm.at[pl.ds(e0 + k0 * _IW, _IW)], gbuf0, gl0)
        if npair == 2:
            dl1 = pltpu.async_copy(g_hbm.at[pl.ds(e0 + (k0 + 1) * _IW, _IW)],
                                   gbuf1, gl1)
        dl0.wait()
        dsc0 = pltpu.async_copy(gbuf0, g_sh.at[idx_d.at[0]], gs0, add=True)
        if npair == 2:
            dl1.wait()
            dsc1 = pltpu.async_copy(gbuf1, g_sh.at[idx_d.at[1]], gs1, add=True)
        dsc0.wait()
        if npair == 2:
            dsc1.wait()

    @pl.loop(0, nch // 2)
    def _(i):
        chunk_pair(2 * i, 2)

    if nch % 2:
        chunk_pair(nch - 1, 1)

    plsc.subcore_barrier()
    _write_shared(g_sh, gp_hbm, c, s, n)


def _sc_pred_body(ept, a_hbm, b_hbm, p_hbm, src_hbm, dst_hbm, w2_hbm, out_hbm,
                  idx_s, idx_d, ab0, bb0, pb0, ab1, bb1, pb1, obuf, tmp, w2v,
                  sa0, sb0, sp0, sa1, sb1, sp1):
    c, s, w = _tile_ids()
    nch = ept // _IW
    pltpu.sync_copy(w2_hbm, w2v)
    p2bv = w2v[pl.ds(128, 16)]
    lane = lax.iota(jnp.int32, 16)
    zlane = jnp.zeros((16,), jnp.int32)

    r0 = w * nch
    e0 = w * ept

    def start(k, j, ab, bb, pb, sema, semb, semp):
        da = pltpu.async_copy(a_hbm.at[idx_s.at[j]], ab, sema)
        db = pltpu.async_copy(b_hbm.at[idx_d.at[j]], bb, semb)
        dp = pltpu.async_copy(p_hbm.at[pl.ds(e0 + k * _IW, _IW)], pb, semp)
        return da, db, dp

    def compute(k, ab, bb, pb):
        @pl.loop(0, _IW)
        def _(e):
            acc = jnp.zeros((16,), _F32)
            for j in range(8):
                sl = pl.ds(j * 16, 16)
                u = ab[e, sl] + bb[e, sl] + pb[e, sl]
                u = jnp.maximum(u, 0.0)
                acc = acc + u * w2v[sl]
            tmp[e, :] = jnp.full((16,), jnp.sum(acc), _F32)

        @pl.loop(0, _IW // 16)
        def _(i):
            t = plsc.load_gather(tmp, [i * 16 + lane, zlane]) + p2bv
            obuf[pl.ds(k * _IW + i * 16, 16)] = 1.0 / (1.0 + jnp.exp(-t))

    @pl.loop(0, nch // 2)
    def _(i):
        k0 = 2 * i
        pltpu.sync_copy(src_hbm.at[pl.ds(r0 + k0, 2)], idx_s)
        pltpu.sync_copy(dst_hbm.at[pl.ds(r0 + k0, 2)], idx_d)
        d0 = start(k0, 0, ab0, bb0, pb0, sa0, sb0, sp0)
        d1 = start(k0 + 1, 1, ab1, bb1, pb1, sa1, sb1, sp1)
        for d in d0:
            d.wait()
        compute(k0, ab0, bb0, pb0)
        for d in d1:
            d.wait()
        compute(k0 + 1, ab1, bb1, pb1)

    if nch % 2:
        k = nch - 1
        pltpu.sync_copy(src_hbm.at[pl.ds(r0 + k, 1)], idx_s.at[pl.ds(0, 1)])
        pltpu.sync_copy(dst_hbm.at[pl.ds(r0 + k, 1)], idx_d.at[pl.ds(0, 1)])
        for d in start(k, 0, ab0, bb0, pb0, sa0, sb0, sp0):
            d.wait()
        compute(k, ab0, bb0, pb0)

    pltpu.sync_copy(obuf, out_hbm.at[pl.ds(e0, ept)])


def kernel(x, edge_index, edge_trust_score, edge_query_embedding,
           c1_nW, c1_nb, c1_e1W, c1_e1b, c1_e2W, c1_e2b,
           c2_nW, c2_nb, c2_e1W, c2_e1b, c2_e2W, c2_e2b,
           p1_W, p1_b, p2_W, p2_b):
    n, din = x.shape
    e = edge_index.shape[1]
    h_dim = c1_nW.shape[1]
    dout = c2_nW.shape[1]
    q = edge_query_embedding.shape[1]
    ept = e // _NW

    src = edge_index[0]
    dst = edge_index[1]
    src2 = src.reshape(e // _IW, _IW)
    dst2 = dst.reshape(e // _IW, _IW)

    def pad_e2(e2w, e2b):
        return jnp.concatenate(
            [e2w, e2b[None, :], jnp.zeros((13, e2w.shape[1]), _F32)], axis=0)

    c1_e2Wp = pad_e2(c1_e2W, c1_e2b)
    c2_e2Wp = pad_e2(c2_e2W, c2_e2b)

    be = 2000
    g_w = 2 * (c1_e1W.shape[1] + 14)
    edge_pre = pl.pallas_call(
        _edge_pre_body,
        grid=(e // be,),
        in_specs=[
            pl.BlockSpec((be, 1), lambda i: (i, 0)),
            pl.BlockSpec((be, q), lambda i: (i, 0)),
            pl.BlockSpec(c1_e1W.shape, lambda i: (0, 0)),
            pl.BlockSpec((1, 34), lambda i: (0, 0)),
            pl.BlockSpec(c2_e1W.shape, lambda i: (0, 0)),
            pl.BlockSpec((1, 34), lambda i: (0, 0)),
            pl.BlockSpec((q, h_dim), lambda i: (0, 0)),
            pl.BlockSpec((1, h_dim), lambda i: (0, 0)),
        ],
        out_specs=[
            pl.BlockSpec((be, g_w), lambda i: (i, 0)),
            pl.BlockSpec((be, h_dim), lambda i: (i, 0)),
        ],
        out_shape=[
            jax.ShapeDtypeStruct((e, g_w), _F32),
            jax.ShapeDtypeStruct((e, h_dim), _F32),
        ],
    )
    gx, p_tab = edge_pre(
        edge_trust_score, edge_query_embedding,
        c1_e1W, c1_e1b.reshape(1, -1), c2_e1W, c2_e1b.reshape(1, -1),
        p1_W[2 * dout:], p1_b.reshape(1, -1))

    mesh = plsc.VectorSubcoreMesh(core_axis_name="core",
                                  subcore_axis_name="subcore")
    sc_params = pltpu.CompilerParams(use_tc_tiling_on_sc=False,
                                     needs_layout_passes=False)
    nch = ept // _IW
    dma = pltpu.SemaphoreType.DMA

    sc_srows = pl.kernel(
        functools.partial(_sc_srows_body, ept, n),
        out_type=jax.ShapeDtypeStruct((2, n, din), _F32),
        mesh=mesh,
        scratch_types=[
            pltpu.VMEM((2, _IW), jnp.int32),
            pltpu.VMEM((2, _IW), jnp.int32),
            pltpu.VMEM((_IW, din), _F32),
            pltpu.VMEM((_IW, din), _F32),
            pltpu.VMEM_SHARED((n, din), _F32),
            dma, dma, dma, dma,
        ],
        compiler_params=sc_params,
    )

    sc_gpack = pl.kernel(
        functools.partial(_sc_gpack_body, ept, n),
        out_type=jax.ShapeDtypeStruct((2, n, g_w), _F32),
        mesh=mesh,
        scratch_types=[
            pltpu.VMEM((2, _IW), jnp.int32),
            pltpu.VMEM((_IW, g_w), _F32),
            pltpu.VMEM((_IW, g_w), _F32),
            pltpu.VMEM_SHARED((n, g_w), _F32),
            dma, dma, dma, dma,
        ],
        compiler_params=sc_params,
    )

    bn = 2000
    conv_update = pl.pallas_call(
        _conv_update_body,
        grid=(n // bn,),
        in_specs=[
            pl.BlockSpec((2, bn, din), lambda i: (0, i, 0)),
            pl.BlockSpec((2, bn, g_w // 2), lambda i: (0, i, 0)),
            pl.BlockSpec((g_w // 2, din), lambda i: (0, 0)),
            pl.BlockSpec((din, h_dim), lambda i: (0, 0)),
            pl.BlockSpec((1, h_dim), lambda i: (0, 0)),
        ],
        out_specs=pl.BlockSpec((bn, h_dim), lambda i: (i, 0)),
        out_shape=jax.ShapeDtypeStruct((n, h_dim), _F32),
    )
    conv_ab = pl.pallas_call(
        _conv_ab_body,
        grid=(n // bn,),
        in_specs=[
            pl.BlockSpec((2, bn, h_dim), lambda i: (0, i, 0)),
            pl.BlockSpec((2, bn, g_w // 2), lambda i: (0, i, 0)),
            pl.BlockSpec((g_w // 2, h_dim), lambda i: (0, 0)),
            pl.BlockSpec((h_dim, dout), lambda i: (0, 0)),
            pl.BlockSpec((1, dout), lambda i: (0, 0)),
            pl.BlockSpec((dout, h_dim), lambda i: (0, 0)),
            pl.BlockSpec((dout, h_dim), lambda i: (0, 0)),
        ],
        out_specs=[
            pl.BlockSpec((bn, h_dim), lambda i: (i, 0)),
            pl.BlockSpec((bn, h_dim), lambda i: (i, 0)),
        ],
        out_shape=[
            jax.ShapeDtypeStruct((n, h_dim), _F32),
            jax.ShapeDtypeStruct((n, h_dim), _F32),
        ],
    )

    sp1 = sc_srows(x, src2, dst2)
    gp = sc_gpack(gx, dst2)
    gp1 = gp[:, :, :g_w // 2]
    gp2 = gp[:, :, g_w // 2:]
    h = conv_update(sp1, gp1, c1_e2Wp, c1_nW, c1_nb.reshape(1, -1))
    sp2 = sc_srows(h, src2, dst2)
    a_tab, b_tab = conv_ab(sp2, gp2, c2_e2Wp, c2_nW, c2_nb.reshape(1, -1),
                           p1_W[:dout], p1_W[dout:2 * dout])

    w2ext = jnp.concatenate(
        [p2_W[:, 0], jnp.full((16,), p2_b[0], _F32)])
    sc_pred = pl.kernel(
        functools.partial(_sc_pred_body, ept),
        out_type=jax.ShapeDtypeStruct((e,), _F32),
        mesh=mesh,
        scratch_types=[
            pltpu.VMEM((2, _IW), jnp.int32),
            pltpu.VMEM((2, _IW), jnp.int32),
            pltpu.VMEM((_IW, h_dim), _F32),
            pltpu.VMEM((_IW, h_dim), _F32),
            pltpu.VMEM((_IW, h_dim), _F32),
            pltpu.VMEM((_IW, h_dim), _F32),
            pltpu.VMEM((_IW, h_dim), _F32),
            pltpu.VMEM((_IW, h_dim), _F32),
            pltpu.VMEM((ept,), _F32),
            pltpu.VMEM((_IW, 16), _F32),
            pltpu.VMEM((144,), _F32),
            dma, dma, dma, dma, dma, dma,
        ],
        compiler_params=sc_params,
    )
    out = sc_pred(a_tab, b_tab, p_tab, src2, dst2, w2ext)
    return out.reshape(e, 1)


# speedup vs baseline: 2.7571x; 1.0268x over previous
import functools

import jax
import jax.numpy as jnp
from jax import lax
from jax.experimental import pallas as pl
from jax.experimental.pallas import tpu as pltpu
from jax.experimental.pallas import tpu_sc as plsc

_F32 = jnp.float32

_IW = 80
_NW = 32


def _edge_pre_body(trust_ref, query_ref, w1a_ref, b1a_ref, w1b_ref, b1b_ref,
                   wq_ref, p1b_ref, g_ref, p_ref):
    ea = jnp.concatenate([trust_ref[...], query_ref[...]], axis=1)
    be = ea.shape[0]
    t1 = jnp.maximum(
        jnp.dot(ea, w1a_ref[...], preferred_element_type=_F32) + b1a_ref[...], 0.0)
    t2 = jnp.maximum(
        jnp.dot(ea, w1b_ref[...], preferred_element_type=_F32) + b1b_ref[...], 0.0)
    ones = jnp.ones((be, 1), _F32)
    pad = jnp.zeros((be, 13), _F32)
    g_ref[...] = jnp.concatenate([t1, ones, pad, t2, ones, pad], axis=1)
    p_ref[...] = (
        jnp.dot(query_ref[...], wq_ref[...], preferred_element_type=_F32)
        + p1b_ref[...])


def _conv_update_body(sp_ref, gp_ref, e2w_ref, nw_ref, nb_ref, h_ref):
    s = sp_ref[0] + sp_ref[1]
    g = gp_ref[0] + gp_ref[1]
    aggr = s + jnp.dot(g, e2w_ref[...], preferred_element_type=_F32)
    h_ref[...] = jnp.maximum(
        jnp.dot(aggr, nw_ref[...], preferred_element_type=_F32) + nb_ref[...], 0.0)


def _conv_ab_body(sp_ref, gp_ref, e2w_ref, nw_ref, nb_ref, wsrc_ref, wdst_ref,
                  a_ref, b_ref):
    s = sp_ref[0] + sp_ref[1]
    g = gp_ref[0] + gp_ref[1]
    aggr = s + jnp.dot(g, e2w_ref[...], preferred_element_type=_F32)
    h2 = jnp.maximum(
        jnp.dot(aggr, nw_ref[...], preferred_element_type=_F32) + nb_ref[...], 0.0)
    a_ref[...] = jnp.dot(h2, wsrc_ref[...], preferred_element_type=_F32)
    b_ref[...] = jnp.dot(h2, wdst_ref[...], preferred_element_type=_F32)


def _tile_ids():
    c = lax.axis_index("core")
    s = lax.axis_index("subcore")
    return c, s, c * 16 + s


def _zero_rows(buf, nrows, width16):
    zero16 = jnp.zeros((16,), _F32)

    @pl.loop(0, nrows)
    def _(i):
        for j in range(width16):
            buf[i, pl.ds(j * 16, 16)] = zero16


def _zero_shared(stage, sh, s, n):
    nt = (n // 16) // 8 * 8
    rem = n - nt * 16
    full, part = divmod(nt, _IW)

    @pl.loop(0, full)
    def _(i):
        pltpu.sync_copy(stage.at[pl.ds(0, _IW)],
                        sh.at[pl.ds(s * nt + i * _IW, _IW)])

    if part:
        pltpu.sync_copy(stage.at[pl.ds(0, part)],
                        sh.at[pl.ds(s * nt + full * _IW, part)])
    if rem:
        @pl.when(s == 15)
        def _():
            pltpu.sync_copy(stage.at[pl.ds(0, rem)], sh.at[pl.ds(nt * 16, rem)])


def _write_shared(sh, out_hbm, c, s, n):
    nt = (n // 16) // 8 * 8
    rem = n - nt * 16
    pltpu.sync_copy(sh.at[pl.ds(s * nt, nt)], out_hbm.at[c, pl.ds(s * nt, nt)])
    if rem:
        @pl.when(s == 15)
        def _():
            pltpu.sync_copy(sh.at[pl.ds(nt * 16, rem)],
                            out_hbm.at[c, pl.ds(nt * 16, rem)])


def _sc_srows_body(ept, n, x_hbm, src_hbm, dst_hbm, sp_hbm,
                   idx_s, idx_d, rows0, rows1, s_sh, gs0, gs1, ss0, ss1):
    c, s, w = _tile_ids()
    nch = ept // _IW

    _zero_rows(rows0, _IW, 8)
    _zero_shared(rows0, s_sh, s, n)
    plsc.subcore_barrier()

    r0 = w * nch

    def chunk_pair(k0, npair):
        pltpu.sync_copy(src_hbm.at[pl.ds(r0 + k0, npair)],
                        idx_s.at[pl.ds(0, npair)])
        pltpu.sync_copy(dst_hbm.at[pl.ds(r0 + k0, npair)],
                        idx_d.at[pl.ds(0, npair)])
        dg0 = pltpu.async_copy(x_hbm.at[idx_s.at[0]], rows0, gs0)
        if npair == 2:
            dg1 = pltpu.async_copy(x_hbm.at[idx_s.at[1]], rows1, gs1)
        dg0.wait()
        dsc0 = pltpu.async_copy(rows0, s_sh.at[idx_d.at[0]], ss0, add=True)
        if npair == 2:
            dg1.wait()
            dsc1 = pltpu.async_copy(rows1, s_sh.at[idx_d.at[1]], ss1, add=True)
        dsc0.wait()
        if npair == 2:
            dsc1.wait()

    @pl.loop(0, nch // 2)
    def _(i):
        chunk_pair(2 * i, 2)

    if nch % 2:
        chunk_pair(nch - 1, 1)

    plsc.subcore_barrier()
    _write_shared(s_sh, sp_hbm, c, s, n)


def _sc_gpack_body(ept, n, g_hbm, dst_hbm, gp_hbm,
                   idx_d, gbuf0, gbuf1, g_sh, gl0, gl1, gs0, gs1):
    c, s, w = _tile_ids()
    nch = ept // _IW

    _zero_rows(gbuf0, _IW, 6)
    _zero_shared(gbuf0, g_sh, s, n)
    plsc.subcore_barrier()

    e0 = w * ept

    def chunk_pair(k0, npair):
        pltpu.sync_copy(dst_hbm.at[pl.ds(w * nch + k0, npair)],
                        idx_d.at[pl.ds(0, npair)])
        dl0 = pltpu.async_copy(g_hbm.at[pl.ds(e0 + k0 * _IW, _IW)], gbuf0, gl0)
        if npair == 2:
            dl1 = pltpu.async_copy(g_hbm.at[pl.ds(e0 + (k0 + 1) * _IW, _IW)],
                                   gbuf1, gl1)
        dl0.wait()
        dsc0 = pltpu.async_copy(gbuf0, g_sh.at[idx_d.at[0]], gs0, add=True)
        if npair == 2:
            dl1.wait()
            dsc1 = pltpu.async_copy(gbuf1, g_sh.at[idx_d.at[1]], gs1, add=True)
        dsc0.wait()
        if npair == 2:
            dsc1.wait()

    @pl.loop(0, nch // 2)
    def _(i):
        chunk_pair(2 * i, 2)

    if nch % 2:
        chunk_pair(nch - 1, 1)

    plsc.subcore_barrier()
    _write_shared(g_sh, gp_hbm, c, s, n)


def _sc_pred_body(ept, a_hbm, b_hbm, p_hbm, src_hbm, dst_hbm, w2_hbm, out_hbm,
                  idx_s, idx_d, ab0, bb0, pb0, ab1, bb1, pb1, ab2, bb2, pb2,
                  obuf, tmp, w2v,
                  sa0, sb0, sp0, sa1, sb1, sp1, sa2, sb2, sp2):
    c, s, w = _tile_ids()
    nch = ept // _IW
    pltpu.sync_copy(w2_hbm, w2v)
    p2bv = w2v[pl.ds(128, 16)]
    lane = lax.iota(jnp.int32, 16)
    zlane = jnp.zeros((16,), jnp.int32)

    r0 = w * nch
    e0 = w * ept
    bufs = ((ab0, bb0, pb0, sa0, sb0, sp0),
            (ab1, bb1, pb1, sa1, sb1, sp1),
            (ab2, bb2, pb2, sa2, sb2, sp2))

    def start(k, j):
        ab, bb, pb, sema, semb, semp = bufs[j]
        da = pltpu.async_copy(a_hbm.at[idx_s.at[j]], ab, sema)
        db = pltpu.async_copy(b_hbm.at[idx_d.at[j]], bb, semb)
        dp = pltpu.async_copy(p_hbm.at[pl.ds(e0 + k * _IW, _IW)], pb, semp)
        return (da, db, dp)

    def compute(k, j):
        ab, bb, pb = bufs[j][:3]

        @pl.loop(0, _IW)
        def _(e):
            acc = jnp.zeros((16,), _F32)
            for jj in range(8):
                sl = pl.ds(jj * 16, 16)
                u = ab[e, sl] + bb[e, sl] + pb[e, sl]
                u = jnp.maximum(u, 0.0)
                acc = acc + u * w2v[sl]
            tmp[e, :] = jnp.full((16,), jnp.sum(acc), _F32)

    def sigmoid_pass(k):
        @pl.loop(0, _IW // 16)
        def _(i):
            t = plsc.load_gather(tmp, [i * 16 + lane, zlane]) + p2bv
            obuf[pl.ds(k * _IW + i * 16, 16)] = 1.0 / (1.0 + jnp.exp(-t))

    @pl.loop(0, nch // 3)
    def _(i):
        k0 = 3 * i
        pltpu.sync_copy(src_hbm.at[pl.ds(r0 + k0, 3)], idx_s)
        pltpu.sync_copy(dst_hbm.at[pl.ds(r0 + k0, 3)], idx_d)
        d0 = start(k0, 0)
        d1 = start(k0 + 1, 1)
        d2 = start(k0 + 2, 2)
        for d in d0:
            d.wait()
        compute(k0, 0)
        sigmoid_pass(k0)
        for d in d1:
            d.wait()
        compute(k0 + 1, 1)
        sigmoid_pass(k0 + 1)
        for d in d2:
            d.wait()
        compute(k0 + 2, 2)
        sigmoid_pass(k0 + 2)

    for k in range(nch // 3 * 3, nch):
        j = k - nch // 3 * 3
        pltpu.sync_copy(src_hbm.at[pl.ds(r0 + k, 1)], idx_s.at[pl.ds(j, 1)])
        pltpu.sync_copy(dst_hbm.at[pl.ds(r0 + k, 1)], idx_d.at[pl.ds(j, 1)])
        for d in start(k, j):
            d.wait()
        compute(k, j)
        sigmoid_pass(k)

    pltpu.sync_copy(obuf, out_hbm.at[pl.ds(e0, ept)])


def kernel(x, edge_index, edge_trust_score, edge_query_embedding,
           c1_nW, c1_nb, c1_e1W, c1_e1b, c1_e2W, c1_e2b,
           c2_nW, c2_nb, c2_e1W, c2_e1b, c2_e2W, c2_e2b,
           p1_W, p1_b, p2_W, p2_b):
    n, din = x.shape
    e = edge_index.shape[1]
    h_dim = c1_nW.shape[1]
    dout = c2_nW.shape[1]
    q = edge_query_embedding.shape[1]
    ept = e // _NW

    src = edge_index[0]
    dst = edge_index[1]
    src2 = src.reshape(e // _IW, _IW)
    dst2 = dst.reshape(e // _IW, _IW)

    def pad_e2(e2w, e2b):
        return jnp.concatenate(
            [e2w, e2b[None, :], jnp.zeros((13, e2w.shape[1]), _F32)], axis=0)

    c1_e2Wp = pad_e2(c1_e2W, c1_e2b)
    c2_e2Wp = pad_e2(c2_e2W, c2_e2b)

    be = 2000
    g_w = 2 * (c1_e1W.shape[1] + 14)
    edge_pre = pl.pallas_call(
        _edge_pre_body,
        grid=(e // be,),
        in_specs=[
            pl.BlockSpec((be, 1), lambda i: (i, 0)),
            pl.BlockSpec((be, q), lambda i: (i, 0)),
            pl.BlockSpec(c1_e1W.shape, lambda i: (0, 0)),
            pl.BlockSpec((1, 34), lambda i: (0, 0)),
            pl.BlockSpec(c2_e1W.shape, lambda i: (0, 0)),
            pl.BlockSpec((1, 34), lambda i: (0, 0)),
            pl.BlockSpec((q, h_dim), lambda i: (0, 0)),
            pl.BlockSpec((1, h_dim), lambda i: (0, 0)),
        ],
        out_specs=[
            pl.BlockSpec((be, g_w), lambda i: (i, 0)),
            pl.BlockSpec((be, h_dim), lambda i: (i, 0)),
        ],
        out_shape=[
            jax.ShapeDtypeStruct((e, g_w), _F32),
            jax.ShapeDtypeStruct((e, h_dim), _F32),
        ],
    )
    gx, p_tab = edge_pre(
        edge_trust_score, edge_query_embedding,
        c1_e1W, c1_e1b.reshape(1, -1), c2_e1W, c2_e1b.reshape(1, -1),
        p1_W[2 * dout:], p1_b.reshape(1, -1))

    mesh = plsc.VectorSubcoreMesh(core_axis_name="core",
                                  subcore_axis_name="subcore")
    sc_params = pltpu.CompilerParams(use_tc_tiling_on_sc=False,
                                     needs_layout_passes=False)
    nch = ept // _IW
    dma = pltpu.SemaphoreType.DMA

    sc_srows = pl.kernel(
        functools.partial(_sc_srows_body, ept, n),
        out_type=jax.ShapeDtypeStruct((2, n, din), _F32),
        mesh=mesh,
        scratch_types=[
            pltpu.VMEM((2, _IW), jnp.int32),
            pltpu.VMEM((2, _IW), jnp.int32),
            pltpu.VMEM((_IW, din), _F32),
            pltpu.VMEM((_IW, din), _F32),
            pltpu.VMEM_SHARED((n, din), _F32),
            dma, dma, dma, dma,
        ],
        compiler_params=sc_params,
    )

    sc_gpack = pl.kernel(
        functools.partial(_sc_gpack_body, ept, n),
        out_type=jax.ShapeDtypeStruct((2, n, g_w), _F32),
        mesh=mesh,
        scratch_types=[
            pltpu.VMEM((2, _IW), jnp.int32),
            pltpu.VMEM((_IW, g_w), _F32),
            pltpu.VMEM((_IW, g_w), _F32),
            pltpu.VMEM_SHARED((n, g_w), _F32),
            dma, dma, dma, dma,
        ],
        compiler_params=sc_params,
    )

    bn = 2000
    conv_update = pl.pallas_call(
        _conv_update_body,
        grid=(n // bn,),
        in_specs=[
            pl.BlockSpec((2, bn, din), lambda i: (0, i, 0)),
            pl.BlockSpec((2, bn, g_w // 2), lambda i: (0, i, 0)),
            pl.BlockSpec((g_w // 2, din), lambda i: (0, 0)),
            pl.BlockSpec((din, h_dim), lambda i: (0, 0)),
            pl.BlockSpec((1, h_dim), lambda i: (0, 0)),
        ],
        out_specs=pl.BlockSpec((bn, h_dim), lambda i: (i, 0)),
        out_shape=jax.ShapeDtypeStruct((n, h_dim), _F32),
    )
    conv_ab = pl.pallas_call(
        _conv_ab_body,
        grid=(n // bn,),
        in_specs=[
            pl.BlockSpec((2, bn, h_dim), lambda i: (0, i, 0)),
            pl.BlockSpec((2, bn, g_w // 2), lambda i: (0, i, 0)),
            pl.BlockSpec((g_w // 2, h_dim), lambda i: (0, 0)),
            pl.BlockSpec((h_dim, dout), lambda i: (0, 0)),
            pl.BlockSpec((1, dout), lambda i: (0, 0)),
            pl.BlockSpec((dout, h_dim), lambda i: (0, 0)),
            pl.BlockSpec((dout, h_dim), lambda i: (0, 0)),
        ],
        out_specs=[
            pl.BlockSpec((bn, h_dim), lambda i: (i, 0)),
            pl.BlockSpec((bn, h_dim), lambda i: (i, 0)),
        ],
        out_shape=[
            jax.ShapeDtypeStruct((n, h_dim), _F32),
            jax.ShapeDtypeStruct((n, h_dim), _F32),
        ],
    )

    sp1 = sc_srows(x, src2, dst2)
    gp = sc_gpack(gx, dst2)
    gp1 = gp[:, :, :g_w // 2]
    gp2 = gp[:, :, g_w // 2:]
    h = conv_update(sp1, gp1, c1_e2Wp, c1_nW, c1_nb.reshape(1, -1))
    sp2 = sc_srows(h, src2, dst2)
    a_tab, b_tab = conv_ab(sp2, gp2, c2_e2Wp, c2_nW, c2_nb.reshape(1, -1),
                           p1_W[:dout], p1_W[dout:2 * dout])

    w2ext = jnp.concatenate(
        [p2_W[:, 0], jnp.full((16,), p2_b[0], _F32)])
    sc_pred = pl.kernel(
        functools.partial(_sc_pred_body, ept),
        out_type=jax.ShapeDtypeStruct((e,), _F32),
        mesh=mesh,
        scratch_types=(
            [pltpu.VMEM((3, _IW), jnp.int32)] * 2
            + [pltpu.VMEM((_IW, h_dim), _F32)] * 9
            + [pltpu.VMEM((ept,), _F32), pltpu.VMEM((_IW, 16), _F32),
               pltpu.VMEM((144,), _F32)]
            + [dma] * 9
        ),
        compiler_params=sc_params,
    )
    out = sc_pred(a_tab, b_tab, p_tab, src2, dst2, w2ext)
    return out.reshape(e, 1)


# speedup vs baseline: 2.8688x; 1.0405x over previous
import functools

import jax
import jax.numpy as jnp
from jax import lax
from jax.experimental import pallas as pl
from jax.experimental.pallas import tpu as pltpu
from jax.experimental.pallas import tpu_sc as plsc

_F32 = jnp.float32

_IW = 80
_NW = 32


def _edge_pre_body(trust_ref, query_ref, w1a_ref, b1a_ref, w1b_ref, b1b_ref,
                   wq_ref, p1b_ref, g_ref, p_ref):
    ea = jnp.concatenate([trust_ref[...], query_ref[...]], axis=1)
    be = ea.shape[0]
    t1 = jnp.maximum(
        jnp.dot(ea, w1a_ref[...], preferred_element_type=_F32) + b1a_ref[...], 0.0)
    t2 = jnp.maximum(
        jnp.dot(ea, w1b_ref[...], preferred_element_type=_F32) + b1b_ref[...], 0.0)
    ones = jnp.ones((be, 1), _F32)
    pad = jnp.zeros((be, 13), _F32)
    g_ref[...] = jnp.concatenate([t1, ones, pad, t2, ones, pad], axis=1)
    p_ref[...] = (
        jnp.dot(query_ref[...], wq_ref[...], preferred_element_type=_F32)
        + p1b_ref[...])


def _conv_update_body(sp_ref, gp_ref, e2w_ref, nw_ref, nb_ref, h_ref):
    s = sp_ref[0] + sp_ref[1]
    g = gp_ref[0] + gp_ref[1]
    aggr = s + jnp.dot(g, e2w_ref[...], preferred_element_type=_F32)
    h_ref[...] = jnp.maximum(
        jnp.dot(aggr, nw_ref[...], preferred_element_type=_F32) + nb_ref[...], 0.0)


def _conv_ab_body(sp_ref, gp_ref, e2w_ref, nw_ref, nb_ref, wsrc_ref, wdst_ref,
                  a_ref, b_ref):
    s = sp_ref[0] + sp_ref[1]
    g = gp_ref[0] + gp_ref[1]
    aggr = s + jnp.dot(g, e2w_ref[...], preferred_element_type=_F32)
    h2 = jnp.maximum(
        jnp.dot(aggr, nw_ref[...], preferred_element_type=_F32) + nb_ref[...], 0.0)
    a_ref[...] = jnp.dot(h2, wsrc_ref[...], preferred_element_type=_F32)
    b_ref[...] = jnp.dot(h2, wdst_ref[...], preferred_element_type=_F32)


def _tile_ids():
    c = lax.axis_index("core")
    s = lax.axis_index("subcore")
    return c, s, c * 16 + s


def _zero_rows(buf, nrows, width16):
    zero16 = jnp.zeros((16,), _F32)

    @pl.loop(0, nrows)
    def _(i):
        for j in range(width16):
            buf[i, pl.ds(j * 16, 16)] = zero16


def _zero_shared(stage, sh, s, n):
    nt = (n // 16) // 8 * 8
    rem = n - nt * 16
    full, part = divmod(nt, _IW)

    @pl.loop(0, full)
    def _(i):
        pltpu.sync_copy(stage.at[pl.ds(0, _IW)],
                        sh.at[pl.ds(s * nt + i * _IW, _IW)])

    if part:
        pltpu.sync_copy(stage.at[pl.ds(0, part)],
                        sh.at[pl.ds(s * nt + full * _IW, part)])
    if rem:
        @pl.when(s == 15)
        def _():
            pltpu.sync_copy(stage.at[pl.ds(0, rem)], sh.at[pl.ds(nt * 16, rem)])


def _write_shared(sh, out_hbm, c, s, n):
    nt = (n // 16) // 8 * 8
    rem = n - nt * 16
    pltpu.sync_copy(sh.at[pl.ds(s * nt, nt)], out_hbm.at[c, pl.ds(s * nt, nt)])
    if rem:
        @pl.when(s == 15)
        def _():
            pltpu.sync_copy(sh.at[pl.ds(nt * 16, rem)],
                            out_hbm.at[c, pl.ds(nt * 16, rem)])


def _sc_srows_body(ept, n, x_hbm, src_hbm, dst_hbm, sp_hbm,
                   idx_s, idx_d, rows0, rows1, s_sh, gs0, gs1, ss0, ss1):
    c, s, w = _tile_ids()
    nch = ept // _IW

    _zero_rows(rows0, _IW, 8)
    _zero_shared(rows0, s_sh, s, n)
    plsc.subcore_barrier()

    r0 = w * nch

    def chunk_pair(k0, npair):
        pltpu.sync_copy(src_hbm.at[pl.ds(r0 + k0, npair)],
                        idx_s.at[pl.ds(0, npair)])
        pltpu.sync_copy(dst_hbm.at[pl.ds(r0 + k0, npair)],
                        idx_d.at[pl.ds(0, npair)])
        dg0 = pltpu.async_copy(x_hbm.at[idx_s.at[0]], rows0, gs0)
        if npair == 2:
            dg1 = pltpu.async_copy(x_hbm.at[idx_s.at[1]], rows1, gs1)
        dg0.wait()
        dsc0 = pltpu.async_copy(rows0, s_sh.at[idx_d.at[0]], ss0, add=True)
        if npair == 2:
            dg1.wait()
            dsc1 = pltpu.async_copy(rows1, s_sh.at[idx_d.at[1]], ss1, add=True)
        dsc0.wait()
        if npair == 2:
            dsc1.wait()

    @pl.loop(0, nch // 2)
    def _(i):
        chunk_pair(2 * i, 2)

    if nch % 2:
        chunk_pair(nch - 1, 1)

    plsc.subcore_barrier()
    _write_shared(s_sh, sp_hbm, c, s, n)


def _sc_gpack_body(ept, n, g_hbm, dst_hbm, gp_hbm,
                   idx_d, gbuf0, gbuf1, g_sh, gl0, gl1, gs0, gs1):
    c, s, w = _tile_ids()
    nch = ept // _IW

    _zero_rows(gbuf0, _IW, 6)
    _zero_shared(gbuf0, g_sh, s, n)
    plsc.subcore_barrier()

    e0 = w * ept

    def chunk_pair(k0, npair):
        pltpu.sync_copy(dst_hbm.at[pl.ds(w * nch + k0, npair)],
                        idx_d.at[pl.ds(0, npair)])
        dl0 = pltpu.async_copy(g_hbm.at[pl.ds(e0 + k0 * _IW, _IW)], gbuf0, gl0)
        if npair == 2:
            dl1 = pltpu.async_copy(g_hbm.at[pl.ds(e0 + (k0 + 1) * _IW, _IW)],
                                   gbuf1, gl1)
        dl0.wait()
        dsc0 = pltpu.async_copy(gbuf0, g_sh.at[idx_d.at[0]], gs0, add=True)
        if npair == 2:
            dl1.wait()
            dsc1 = pltpu.async_copy(gbuf1, g_sh.at[idx_d.at[1]], gs1, add=True)
        dsc0.wait()
        if npair == 2:
            dsc1.wait()

    @pl.loop(0, nch // 2)
    def _(i):
        chunk_pair(2 * i, 2)

    if nch % 2:
        chunk_pair(nch - 1, 1)

    plsc.subcore_barrier()
    _write_shared(g_sh, gp_hbm, c, s, n)


def _sc_pred_body(ept, a_hbm, b_hbm, p_hbm, src_hbm, dst_hbm, w2_hbm, out_hbm,
                  idx_s, idx_d, ab0, bb0, pb0, ab1, bb1, pb1, ab2, bb2, pb2,
                  obuf, tmp, w2v,
                  sa0, sb0, sp0, sa1, sb1, sp1, sa2, sb2, sp2):
    c, s, w = _tile_ids()
    nch = ept // _IW
    pltpu.sync_copy(w2_hbm, w2v)
    p2bv = w2v[pl.ds(128, 16)]
    lane = lax.iota(jnp.int32, 16)
    zlane = jnp.zeros((16,), jnp.int32)

    r0 = w * nch
    e0 = w * ept
    bufs = ((ab0, bb0, pb0, sa0, sb0, sp0),
            (ab1, bb1, pb1, sa1, sb1, sp1),
            (ab2, bb2, pb2, sa2, sb2, sp2))

    def start(k, j):
        ab, bb, pb, sema, semb, semp = bufs[j]
        da = pltpu.async_copy(a_hbm.at[idx_s.at[j]], ab, sema)
        db = pltpu.async_copy(b_hbm.at[idx_d.at[j]], bb, semb)
        dp = pltpu.async_copy(p_hbm.at[pl.ds(e0 + k * _IW, _IW)], pb, semp)
        return (da, db, dp)

    def compute(k, j):
        ab, bb, pb = bufs[j][:3]

        @pl.loop(0, _IW, step=2)
        def _(e):
            acc0 = jnp.zeros((16,), _F32)
            acc1 = jnp.zeros((16,), _F32)
            for jj in range(8):
                sl = pl.ds(jj * 16, 16)
                w2c = w2v[sl]
                u0 = ab[e, sl] + bb[e, sl] + pb[e, sl]
                u1 = ab[e + 1, sl] + bb[e + 1, sl] + pb[e + 1, sl]
                acc0 = acc0 + jnp.maximum(u0, 0.0) * w2c
                acc1 = acc1 + jnp.maximum(u1, 0.0) * w2c
            tmp[e, :] = jnp.full((16,), jnp.sum(acc0), _F32)
            tmp[e + 1, :] = jnp.full((16,), jnp.sum(acc1), _F32)

    def sigmoid_pass(k):
        @pl.loop(0, _IW // 16)
        def _(i):
            t = plsc.load_gather(tmp, [i * 16 + lane, zlane]) + p2bv
            obuf[pl.ds(k * _IW + i * 16, 16)] = 1.0 / (1.0 + jnp.exp(-t))

    @pl.loop(0, nch // 3)
    def _(i):
        k0 = 3 * i
        pltpu.sync_copy(src_hbm.at[pl.ds(r0 + k0, 3)], idx_s)
        pltpu.sync_copy(dst_hbm.at[pl.ds(r0 + k0, 3)], idx_d)
        d0 = start(k0, 0)
        d1 = start(k0 + 1, 1)
        d2 = start(k0 + 2, 2)
        for d in d0:
            d.wait()
        compute(k0, 0)
        sigmoid_pass(k0)
        for d in d1:
            d.wait()
        compute(k0 + 1, 1)
        sigmoid_pass(k0 + 1)
        for d in d2:
            d.wait()
        compute(k0 + 2, 2)
        sigmoid_pass(k0 + 2)

    for k in range(nch // 3 * 3, nch):
        j = k - nch // 3 * 3
        pltpu.sync_copy(src_hbm.at[pl.ds(r0 + k, 1)], idx_s.at[pl.ds(j, 1)])
        pltpu.sync_copy(dst_hbm.at[pl.ds(r0 + k, 1)], idx_d.at[pl.ds(j, 1)])
        for d in start(k, j):
            d.wait()
        compute(k, j)
        sigmoid_pass(k)

    pltpu.sync_copy(obuf, out_hbm.at[pl.ds(e0, ept)])


def kernel(x, edge_index, edge_trust_score, edge_query_embedding,
           c1_nW, c1_nb, c1_e1W, c1_e1b, c1_e2W, c1_e2b,
           c2_nW, c2_nb, c2_e1W, c2_e1b, c2_e2W, c2_e2b,
           p1_W, p1_b, p2_W, p2_b):
    n, din = x.shape
    e = edge_index.shape[1]
    h_dim = c1_nW.shape[1]
    dout = c2_nW.shape[1]
    q = edge_query_embedding.shape[1]
    ept = e // _NW

    src = edge_index[0]
    dst = edge_index[1]
    src2 = src.reshape(e // _IW, _IW)
    dst2 = dst.reshape(e // _IW, _IW)

    def pad_e2(e2w, e2b):
        return jnp.concatenate(
            [e2w, e2b[None, :], jnp.zeros((13, e2w.shape[1]), _F32)], axis=0)

    c1_e2Wp = pad_e2(c1_e2W, c1_e2b)
    c2_e2Wp = pad_e2(c2_e2W, c2_e2b)

    be = 2000
    g_w = 2 * (c1_e1W.shape[1] + 14)
    edge_pre = pl.pallas_call(
        _edge_pre_body,
        grid=(e // be,),
        in_specs=[
            pl.BlockSpec((be, 1), lambda i: (i, 0)),
            pl.BlockSpec((be, q), lambda i: (i, 0)),
            pl.BlockSpec(c1_e1W.shape, lambda i: (0, 0)),
            pl.BlockSpec((1, 34), lambda i: (0, 0)),
            pl.BlockSpec(c2_e1W.shape, lambda i: (0, 0)),
            pl.BlockSpec((1, 34), lambda i: (0, 0)),
            pl.BlockSpec((q, h_dim), lambda i: (0, 0)),
            pl.BlockSpec((1, h_dim), lambda i: (0, 0)),
        ],
        out_specs=[
            pl.BlockSpec((be, g_w), lambda i: (i, 0)),
            pl.BlockSpec((be, h_dim), lambda i: (i, 0)),
        ],
        out_shape=[
            jax.ShapeDtypeStruct((e, g_w), _F32),
            jax.ShapeDtypeStruct((e, h_dim), _F32),
        ],
    )
    gx, p_tab = edge_pre(
        edge_trust_score, edge_query_embedding,
        c1_e1W, c1_e1b.reshape(1, -1), c2_e1W, c2_e1b.reshape(1, -1),
        p1_W[2 * dout:], p1_b.reshape(1, -1))

    mesh = plsc.VectorSubcoreMesh(core_axis_name="core",
                                  subcore_axis_name="subcore")
    sc_params = pltpu.CompilerParams(use_tc_tiling_on_sc=False,
                                     needs_layout_passes=False)
    nch = ept // _IW
    dma = pltpu.SemaphoreType.DMA

    sc_srows = pl.kernel(
        functools.partial(_sc_srows_body, ept, n),
        out_type=jax.ShapeDtypeStruct((2, n, din), _F32),
        mesh=mesh,
        scratch_types=[
            pltpu.VMEM((2, _IW), jnp.int32),
            pltpu.VMEM((2, _IW), jnp.int32),
            pltpu.VMEM((_IW, din), _F32),
            pltpu.VMEM((_IW, din), _F32),
            pltpu.VMEM_SHARED((n, din), _F32),
            dma, dma, dma, dma,
        ],
        compiler_params=sc_params,
    )

    sc_gpack = pl.kernel(
        functools.partial(_sc_gpack_body, ept, n),
        out_type=jax.ShapeDtypeStruct((2, n, g_w), _F32),
        mesh=mesh,
        scratch_types=[
            pltpu.VMEM((2, _IW), jnp.int32),
            pltpu.VMEM((_IW, g_w), _F32),
            pltpu.VMEM((_IW, g_w), _F32),
            pltpu.VMEM_SHARED((n, g_w), _F32),
            dma, dma, dma, dma,
        ],
        compiler_params=sc_params,
    )

    bn = 2000
    conv_update = pl.pallas_call(
        _conv_update_body,
        grid=(n // bn,),
        in_specs=[
            pl.BlockSpec((2, bn, din), lambda i: (0, i, 0)),
            pl.BlockSpec((2, bn, g_w // 2), lambda i: (0, i, 0)),
            pl.BlockSpec((g_w // 2, din), lambda i: (0, 0)),
            pl.BlockSpec((din, h_dim), lambda i: (0, 0)),
            pl.BlockSpec((1, h_dim), lambda i: (0, 0)),
        ],
        out_specs=pl.BlockSpec((bn, h_dim), lambda i: (i, 0)),
        out_shape=jax.ShapeDtypeStruct((n, h_dim), _F32),
    )
    conv_ab = pl.pallas_call(
        _conv_ab_body,
        grid=(n // bn,),
        in_specs=[
            pl.BlockSpec((2, bn, h_dim), lambda i: (0, i, 0)),
            pl.BlockSpec((2, bn, g_w // 2), lambda i: (0, i, 0)),
            pl.BlockSpec((g_w // 2, h_dim), lambda i: (0, 0)),
            pl.BlockSpec((h_dim, dout), lambda i: (0, 0)),
            pl.BlockSpec((1, dout), lambda i: (0, 0)),
            pl.BlockSpec((dout, h_dim), lambda i: (0, 0)),
            pl.BlockSpec((dout, h_dim), lambda i: (0, 0)),
        ],
        out_specs=[
            pl.BlockSpec((bn, h_dim), lambda i: (i, 0)),
            pl.BlockSpec((bn, h_dim), lambda i: (i, 0)),
        ],
        out_shape=[
            jax.ShapeDtypeStruct((n, h_dim), _F32),
            jax.ShapeDtypeStruct((n, h_dim), _F32),
        ],
    )

    sp1 = sc_srows(x, src2, dst2)
    gp = sc_gpack(gx, dst2)
    gp1 = gp[:, :, :g_w // 2]
    gp2 = gp[:, :, g_w // 2:]
    h = conv_update(sp1, gp1, c1_e2Wp, c1_nW, c1_nb.reshape(1, -1))
    sp2 = sc_srows(h, src2, dst2)
    a_tab, b_tab = conv_ab(sp2, gp2, c2_e2Wp, c2_nW, c2_nb.reshape(1, -1),
                           p1_W[:dout], p1_W[dout:2 * dout])

    w2ext = jnp.concatenate(
        [p2_W[:, 0], jnp.full((16,), p2_b[0], _F32)])
    sc_pred = pl.kernel(
        functools.partial(_sc_pred_body, ept),
        out_type=jax.ShapeDtypeStruct((e,), _F32),
        mesh=mesh,
        scratch_types=(
            [pltpu.VMEM((3, _IW), jnp.int32)] * 2
            + [pltpu.VMEM((_IW, h_dim), _F32)] * 9
            + [pltpu.VMEM((ept,), _F32), pltpu.VMEM((_IW, 16), _F32),
               pltpu.VMEM((144,), _F32)]
            + [dma] * 9
        ),
        compiler_params=sc_params,
    )
    out = sc_pred(a_tab, b_tab, p_tab, src2, dst2, w2ext)
    return out.reshape(e, 1)


# speedup vs baseline: 3.2335x; 1.1271x over previous
import functools

import jax
import jax.numpy as jnp
from jax import lax
from jax.experimental import pallas as pl
from jax.experimental.pallas import tpu as pltpu
from jax.experimental.pallas import tpu_sc as plsc

_F32 = jnp.float32

_IW = 80
_NW = 32


def _edge_pre_body(trust_ref, query_ref, w1a_ref, b1a_ref, w1b_ref, b1b_ref,
                   wq_ref, p1b_ref, g_ref, p_ref):
    ea = jnp.concatenate([trust_ref[...], query_ref[...]], axis=1)
    be = ea.shape[0]
    t1 = jnp.maximum(
        jnp.dot(ea, w1a_ref[...], preferred_element_type=_F32) + b1a_ref[...], 0.0)
    t2 = jnp.maximum(
        jnp.dot(ea, w1b_ref[...], preferred_element_type=_F32) + b1b_ref[...], 0.0)
    ones = jnp.ones((be, 1), _F32)
    pad = jnp.zeros((be, 13), _F32)
    g_ref[...] = jnp.concatenate([t1, ones, pad, t2, ones, pad], axis=1)
    p_ref[...] = (
        jnp.dot(query_ref[...], wq_ref[...], preferred_element_type=_F32)
        + p1b_ref[...])


def _conv_update_body(sp_ref, gp_ref, e2w_ref, nw_ref, nb_ref, h_ref):
    s = sp_ref[0] + sp_ref[1]
    g = gp_ref[0] + gp_ref[1]
    aggr = s + jnp.dot(g, e2w_ref[...], preferred_element_type=_F32)
    h_ref[...] = jnp.maximum(
        jnp.dot(aggr, nw_ref[...], preferred_element_type=_F32) + nb_ref[...], 0.0)


def _conv_ab_body(sp_ref, gp_ref, e2w_ref, nw_ref, nb_ref, wsrc_ref, wdst_ref,
                  a_ref, b_ref):
    s = sp_ref[0] + sp_ref[1]
    g = gp_ref[0] + gp_ref[1]
    aggr = s + jnp.dot(g, e2w_ref[...], preferred_element_type=_F32)
    h2 = jnp.maximum(
        jnp.dot(aggr, nw_ref[...], preferred_element_type=_F32) + nb_ref[...], 0.0)
    a_ref[...] = jnp.dot(h2, wsrc_ref[...], preferred_element_type=_F32)
    b_ref[...] = jnp.dot(h2, wdst_ref[...], preferred_element_type=_F32)


def _tile_ids():
    c = lax.axis_index("core")
    s = lax.axis_index("subcore")
    return c, s, c * 16 + s


def _zero_rows(buf, nrows, width16):
    zero16 = jnp.zeros((16,), _F32)

    @pl.loop(0, nrows)
    def _(i):
        for j in range(width16):
            buf[i, pl.ds(j * 16, 16)] = zero16


def _zero_shared(stage, sh, s, n):
    nt = (n // 16) // 8 * 8
    rem = n - nt * 16
    full, part = divmod(nt, _IW)

    @pl.loop(0, full)
    def _(i):
        pltpu.sync_copy(stage.at[pl.ds(0, _IW)],
                        sh.at[pl.ds(s * nt + i * _IW, _IW)])

    if part:
        pltpu.sync_copy(stage.at[pl.ds(0, part)],
                        sh.at[pl.ds(s * nt + full * _IW, part)])
    if rem:
        @pl.when(s == 15)
        def _():
            pltpu.sync_copy(stage.at[pl.ds(0, rem)], sh.at[pl.ds(nt * 16, rem)])


def _write_shared(sh, out_hbm, c, s, n):
    nt = (n // 16) // 8 * 8
    rem = n - nt * 16
    pltpu.sync_copy(sh.at[pl.ds(s * nt, nt)], out_hbm.at[c, pl.ds(s * nt, nt)])
    if rem:
        @pl.when(s == 15)
        def _():
            pltpu.sync_copy(sh.at[pl.ds(nt * 16, rem)],
                            out_hbm.at[c, pl.ds(nt * 16, rem)])


def _scatter_pipeline(w, nch, idx_s, idx_d, bufs, sh, gsems, ssems,
                      load_chunk, src_hbm, dst_hbm):
    r0 = w * nch

    def load_idx(q, half):
        if src_hbm is not None:
            pltpu.sync_copy(src_hbm.at[pl.ds(r0 + 4 * q, 4)],
                            idx_s.at[pl.ds(4 * half, 4)])
        pltpu.sync_copy(dst_hbm.at[pl.ds(r0 + 4 * q, 4)],
                        idx_d.at[pl.ds(4 * half, 4)])

    def scat(t, half):
        return pltpu.async_copy(bufs[t], sh.at[idx_d.at[4 * half + t]],
                                ssems[t], add=True)

    def wait_scat(t, half):
        pltpu.make_async_copy(bufs[t], sh.at[idx_d.at[4 * half + t]],
                              ssems[t]).wait()

    def quadstep(q, half, first):
        load_idx(q, half)
        dgs = []
        for t in range(4):
            if not first:
                wait_scat(t, 1 - half)
            dgs.append(load_chunk(4 * q + t, 4 * half + t, bufs[t], gsems[t]))
        for t in range(4):
            dgs[t].wait()
            scat(t, half)

    nquad = nch // 4
    @pl.loop(0, nquad // 2)
    def _(m):
        @pl.when(m == 0)
        def _():
            quadstep(0, 0, True)

        @pl.when(m > 0)
        def _():
            quadstep(2 * m, 0, False)
        quadstep(2 * m + 1, 1, False)

    rem_q = nquad // 2 * 2
    for q in range(rem_q, nquad):
        quadstep(q, 0, False)
    last_half = 0 if rem_q < nquad else 1
    for t in range(4):
        wait_scat(t, last_half)
    for k in range(nquad * 4, nch):
        if src_hbm is not None:
            pltpu.sync_copy(src_hbm.at[pl.ds(r0 + k, 1)], idx_s.at[pl.ds(0, 1)])
        pltpu.sync_copy(dst_hbm.at[pl.ds(r0 + k, 1)], idx_d.at[pl.ds(0, 1)])
        load_chunk(k, 0, bufs[0], gsems[0]).wait()
        scat(0, 0).wait()


def _sc_srows_body(ept, n, x_hbm, src_hbm, dst_hbm, sp_hbm,
                   idx_s, idx_d, rows0, rows1, rows2, rows3, s_sh,
                   gs0, gs1, gs2, gs3, ss0, ss1, ss2, ss3):
    c, s, w = _tile_ids()
    nch = ept // _IW

    _zero_rows(rows0, _IW, 8)
    _zero_shared(rows0, s_sh, s, n)
    plsc.subcore_barrier()

    def load_chunk(k, idx_row, buf, sem):
        return pltpu.async_copy(x_hbm.at[idx_s.at[idx_row]], buf, sem)

    _scatter_pipeline(w, nch, idx_s, idx_d, (rows0, rows1, rows2, rows3),
                      s_sh, (gs0, gs1, gs2, gs3), (ss0, ss1, ss2, ss3),
                      load_chunk, src_hbm, dst_hbm)

    plsc.subcore_barrier()
    _write_shared(s_sh, sp_hbm, c, s, n)


def _sc_gpack_body(ept, n, g_hbm, dst_hbm, gp_hbm,
                   idx_d, gbuf0, gbuf1, gbuf2, gbuf3, g_sh,
                   gl0, gl1, gl2, gl3, gs0, gs1, gs2, gs3):
    c, s, w = _tile_ids()
    nch = ept // _IW

    _zero_rows(gbuf0, _IW, 6)
    _zero_shared(gbuf0, g_sh, s, n)
    plsc.subcore_barrier()

    e0 = w * ept

    def load_chunk(k, idx_row, buf, sem):
        return pltpu.async_copy(
            g_hbm.at[pl.ds(e0 + k * _IW, _IW)], buf, sem)

    _scatter_pipeline(w, nch, None, idx_d, (gbuf0, gbuf1, gbuf2, gbuf3),
                      g_sh, (gl0, gl1, gl2, gl3), (gs0, gs1, gs2, gs3),
                      load_chunk, None, dst_hbm)

    plsc.subcore_barrier()
    _write_shared(g_sh, gp_hbm, c, s, n)


def _sc_pred_body(ept, a_hbm, b_hbm, p_hbm, src_hbm, dst_hbm, w2_hbm, out_hbm,
                  idx_s, idx_d, ab0, bb0, pb0, ab1, bb1, pb1, ab2, bb2, pb2,
                  obuf, tmp, w2v,
                  sa0, sb0, sp0, sa1, sb1, sp1, sa2, sb2, sp2):
    c, s, w = _tile_ids()
    nch = ept // _IW
    pltpu.sync_copy(w2_hbm, w2v)
    p2bv = w2v[pl.ds(128, 16)]
    lane = lax.iota(jnp.int32, 16)
    zlane = jnp.zeros((16,), jnp.int32)

    r0 = w * nch
    e0 = w * ept
    bufs = ((ab0, bb0, pb0, sa0, sb0, sp0),
            (ab1, bb1, pb1, sa1, sb1, sp1),
            (ab2, bb2, pb2, sa2, sb2, sp2))

    def start(k, j):
        ab, bb, pb, sema, semb, semp = bufs[j]
        da = pltpu.async_copy(a_hbm.at[idx_s.at[j]], ab, sema)
        db = pltpu.async_copy(b_hbm.at[idx_d.at[j]], bb, semb)
        dp = pltpu.async_copy(p_hbm.at[pl.ds(e0 + k * _IW, _IW)], pb, semp)
        return (da, db, dp)

    def compute(k, j):
        ab, bb, pb = bufs[j][:3]

        @pl.loop(0, _IW, step=2)
        def _(e):
            acc0 = jnp.zeros((16,), _F32)
            acc1 = jnp.zeros((16,), _F32)
            for jj in range(8):
                sl = pl.ds(jj * 16, 16)
                w2c = w2v[sl]
                u0 = ab[e, sl] + bb[e, sl] + pb[e, sl]
                u1 = ab[e + 1, sl] + bb[e + 1, sl] + pb[e + 1, sl]
                acc0 = acc0 + jnp.maximum(u0, 0.0) * w2c
                acc1 = acc1 + jnp.maximum(u1, 0.0) * w2c
            tmp[e, :] = jnp.full((16,), jnp.sum(acc0), _F32)
            tmp[e + 1, :] = jnp.full((16,), jnp.sum(acc1), _F32)

    def sigmoid_pass(k):
        @pl.loop(0, _IW // 16)
        def _(i):
            t = plsc.load_gather(tmp, [i * 16 + lane, zlane]) + p2bv
            obuf[pl.ds(k * _IW + i * 16, 16)] = 1.0 / (1.0 + jnp.exp(-t))

    @pl.loop(0, nch // 3)
    def _(i):
        k0 = 3 * i
        pltpu.sync_copy(src_hbm.at[pl.ds(r0 + k0, 3)], idx_s)
        pltpu.sync_copy(dst_hbm.at[pl.ds(r0 + k0, 3)], idx_d)
        d0 = start(k0, 0)
        d1 = start(k0 + 1, 1)
        d2 = start(k0 + 2, 2)
        for d in d0:
            d.wait()
        compute(k0, 0)
        sigmoid_pass(k0)
        for d in d1:
            d.wait()
        compute(k0 + 1, 1)
        sigmoid_pass(k0 + 1)
        for d in d2:
            d.wait()
        compute(k0 + 2, 2)
        sigmoid_pass(k0 + 2)

    for k in range(nch // 3 * 3, nch):
        j = k - nch // 3 * 3
        pltpu.sync_copy(src_hbm.at[pl.ds(r0 + k, 1)], idx_s.at[pl.ds(j, 1)])
        pltpu.sync_copy(dst_hbm.at[pl.ds(r0 + k, 1)], idx_d.at[pl.ds(j, 1)])
        for d in start(k, j):
            d.wait()
        compute(k, j)
        sigmoid_pass(k)

    pltpu.sync_copy(obuf, out_hbm.at[pl.ds(e0, ept)])


def kernel(x, edge_index, edge_trust_score, edge_query_embedding,
           c1_nW, c1_nb, c1_e1W, c1_e1b, c1_e2W, c1_e2b,
           c2_nW, c2_nb, c2_e1W, c2_e1b, c2_e2W, c2_e2b,
           p1_W, p1_b, p2_W, p2_b):
    n, din = x.shape
    e = edge_index.shape[1]
    h_dim = c1_nW.shape[1]
    dout = c2_nW.shape[1]
    q = edge_query_embedding.shape[1]
    ept = e // _NW

    src = edge_index[0]
    dst = edge_index[1]
    src2 = src.reshape(e // _IW, _IW)
    dst2 = dst.reshape(e // _IW, _IW)

    def pad_e2(e2w, e2b):
        return jnp.concatenate(
            [e2w, e2b[None, :], jnp.zeros((13, e2w.shape[1]), _F32)], axis=0)

    c1_e2Wp = pad_e2(c1_e2W, c1_e2b)
    c2_e2Wp = pad_e2(c2_e2W, c2_e2b)

    be = 2000
    g_w = 2 * (c1_e1W.shape[1] + 14)
    edge_pre = pl.pallas_call(
        _edge_pre_body,
        grid=(e // be,),
        in_specs=[
            pl.BlockSpec((be, 1), lambda i: (i, 0)),
            pl.BlockSpec((be, q), lambda i: (i, 0)),
            pl.BlockSpec(c1_e1W.shape, lambda i: (0, 0)),
            pl.BlockSpec((1, 34), lambda i: (0, 0)),
            pl.BlockSpec(c2_e1W.shape, lambda i: (0, 0)),
            pl.BlockSpec((1, 34), lambda i: (0, 0)),
            pl.BlockSpec((q, h_dim), lambda i: (0, 0)),
            pl.BlockSpec((1, h_dim), lambda i: (0, 0)),
        ],
        out_specs=[
            pl.BlockSpec((be, g_w), lambda i: (i, 0)),
            pl.BlockSpec((be, h_dim), lambda i: (i, 0)),
        ],
        out_shape=[
            jax.ShapeDtypeStruct((e, g_w), _F32),
            jax.ShapeDtypeStruct((e, h_dim), _F32),
        ],
    )
    gx, p_tab = edge_pre(
        edge_trust_score, edge_query_embedding,
        c1_e1W, c1_e1b.reshape(1, -1), c2_e1W, c2_e1b.reshape(1, -1),
        p1_W[2 * dout:], p1_b.reshape(1, -1))

    mesh = plsc.VectorSubcoreMesh(core_axis_name="core",
                                  subcore_axis_name="subcore")
    sc_params = pltpu.CompilerParams(use_tc_tiling_on_sc=False,
                                     needs_layout_passes=False)
    nch = ept // _IW
    dma = pltpu.SemaphoreType.DMA

    sc_srows = pl.kernel(
        functools.partial(_sc_srows_body, ept, n),
        out_type=jax.ShapeDtypeStruct((2, n, din), _F32),
        mesh=mesh,
        scratch_types=(
            [pltpu.VMEM((8, _IW), jnp.int32)] * 2
            + [pltpu.VMEM((_IW, din), _F32)] * 4
            + [pltpu.VMEM_SHARED((n, din), _F32)]
            + [dma] * 8
        ),
        compiler_params=sc_params,
    )

    sc_gpack = pl.kernel(
        functools.partial(_sc_gpack_body, ept, n),
        out_type=jax.ShapeDtypeStruct((2, n, g_w), _F32),
        mesh=mesh,
        scratch_types=(
            [pltpu.VMEM((8, _IW), jnp.int32)]
            + [pltpu.VMEM((_IW, g_w), _F32)] * 4
            + [pltpu.VMEM_SHARED((n, g_w), _F32)]
            + [dma] * 8
        ),
        compiler_params=sc_params,
    )

    bn = 2000
    conv_update = pl.pallas_call(
        _conv_update_body,
        grid=(n // bn,),
        in_specs=[
            pl.BlockSpec((2, bn, din), lambda i: (0, i, 0)),
            pl.BlockSpec((2, bn, g_w // 2), lambda i: (0, i, 0)),
            pl.BlockSpec((g_w // 2, din), lambda i: (0, 0)),
            pl.BlockSpec((din, h_dim), lambda i: (0, 0)),
            pl.BlockSpec((1, h_dim), lambda i: (0, 0)),
        ],
        out_specs=pl.BlockSpec((bn, h_dim), lambda i: (i, 0)),
        out_shape=jax.ShapeDtypeStruct((n, h_dim), _F32),
    )
    conv_ab = pl.pallas_call(
        _conv_ab_body,
        grid=(n // bn,),
        in_specs=[
            pl.BlockSpec((2, bn, h_dim), lambda i: (0, i, 0)),
            pl.BlockSpec((2, bn, g_w // 2), lambda i: (0, i, 0)),
            pl.BlockSpec((g_w // 2, h_dim), lambda i: (0, 0)),
            pl.BlockSpec((h_dim, dout), lambda i: (0, 0)),
            pl.BlockSpec((1, dout), lambda i: (0, 0)),
            pl.BlockSpec((dout, h_dim), lambda i: (0, 0)),
            pl.BlockSpec((dout, h_dim), lambda i: (0, 0)),
        ],
        out_specs=[
            pl.BlockSpec((bn, h_dim), lambda i: (i, 0)),
            pl.BlockSpec((bn, h_dim), lambda i: (i, 0)),
        ],
        out_shape=[
            jax.ShapeDtypeStruct((n, h_dim), _F32),
            jax.ShapeDtypeStruct((n, h_dim), _F32),
        ],
    )

    sp1 = sc_srows(x, src2, dst2)
    gp = sc_gpack(gx, dst2)
    gp1 = gp[:, :, :g_w // 2]
    gp2 = gp[:, :, g_w // 2:]
    h = conv_update(sp1, gp1, c1_e2Wp, c1_nW, c1_nb.reshape(1, -1))
    sp2 = sc_srows(h, src2, dst2)
    a_tab, b_tab = conv_ab(sp2, gp2, c2_e2Wp, c2_nW, c2_nb.reshape(1, -1),
                           p1_W[:dout], p1_W[dout:2 * dout])

    w2ext = jnp.concatenate(
        [p2_W[:, 0], jnp.full((16,), p2_b[0], _F32)])
    sc_pred = pl.kernel(
        functools.partial(_sc_pred_body, ept),
        out_type=jax.ShapeDtypeStruct((e,), _F32),
        mesh=mesh,
        scratch_types=(
            [pltpu.VMEM((3, _IW), jnp.int32)] * 2
            + [pltpu.VMEM((_IW, h_dim), _F32)] * 9
            + [pltpu.VMEM((ept,), _F32), pltpu.VMEM((_IW, 16), _F32),
               pltpu.VMEM((144,), _F32)]
            + [dma] * 9
        ),
        compiler_params=sc_params,
    )
    out = sc_pred(a_tab, b_tab, p_tab, src2, dst2, w2ext)
    return out.reshape(e, 1)


# speedup vs baseline: 3.4424x; 1.0646x over previous
import functools

import jax
import jax.numpy as jnp
from jax import lax
from jax.experimental import pallas as pl
from jax.experimental.pallas import tpu as pltpu
from jax.experimental.pallas import tpu_sc as plsc

_F32 = jnp.float32

_IW = 80
_NW = 32


def _edge_pre_body(trust_ref, query_ref, w1a_ref, b1a_ref, w1b_ref, b1b_ref,
                   wq_ref, p1b_ref, g_ref, p_ref):
    ea = jnp.concatenate([trust_ref[...], query_ref[...]], axis=1)
    be = ea.shape[0]
    t1 = jnp.maximum(
        jnp.dot(ea, w1a_ref[...], preferred_element_type=_F32) + b1a_ref[...], 0.0)
    t2 = jnp.maximum(
        jnp.dot(ea, w1b_ref[...], preferred_element_type=_F32) + b1b_ref[...], 0.0)
    ones = jnp.ones((be, 1), _F32)
    pad = jnp.zeros((be, 13), _F32)
    g_ref[...] = jnp.concatenate([t1, ones, pad, t2, ones, pad], axis=1)
    p_ref[...] = (
        jnp.dot(query_ref[...], wq_ref[...], preferred_element_type=_F32)
        + p1b_ref[...])


def _conv_update_body(sp_ref, gp_ref, e2w_ref, nw_ref, nb_ref, h_ref):
    s = sp_ref[0] + sp_ref[1]
    g = gp_ref[0] + gp_ref[1]
    aggr = s + jnp.dot(g, e2w_ref[...], preferred_element_type=_F32)
    h_ref[...] = jnp.maximum(
        jnp.dot(aggr, nw_ref[...], preferred_element_type=_F32) + nb_ref[...], 0.0)


def _conv_ab_body(sp_ref, gp_ref, e2w_ref, nw_ref, nb_ref, wsrc_ref, wdst_ref,
                  a_ref, b_ref):
    s = sp_ref[0] + sp_ref[1]
    g = gp_ref[0] + gp_ref[1]
    aggr = s + jnp.dot(g, e2w_ref[...], preferred_element_type=_F32)
    h2 = jnp.maximum(
        jnp.dot(aggr, nw_ref[...], preferred_element_type=_F32) + nb_ref[...], 0.0)
    a_ref[...] = jnp.dot(h2, wsrc_ref[...], preferred_element_type=_F32)
    b_ref[...] = jnp.dot(h2, wdst_ref[...], preferred_element_type=_F32)


def _tile_ids():
    c = lax.axis_index("core")
    s = lax.axis_index("subcore")
    return c, s, c * 16 + s


def _zero_rows(buf, nrows, width16):
    zero16 = jnp.zeros((16,), _F32)

    @pl.loop(0, nrows)
    def _(i):
        for j in range(width16):
            buf[i, pl.ds(j * 16, 16)] = zero16


def _zero_shared(stage, sh, s, n):
    nt = (n // 16) // 8 * 8
    rem = n - nt * 16
    full, part = divmod(nt, _IW)

    @pl.loop(0, full)
    def _(i):
        pltpu.sync_copy(stage.at[pl.ds(0, _IW)],
                        sh.at[pl.ds(s * nt + i * _IW, _IW)])

    if part:
        pltpu.sync_copy(stage.at[pl.ds(0, part)],
                        sh.at[pl.ds(s * nt + full * _IW, part)])
    if rem:
        @pl.when(s == 15)
        def _():
            pltpu.sync_copy(stage.at[pl.ds(0, rem)], sh.at[pl.ds(nt * 16, rem)])


def _write_shared(sh, out_hbm, c, s, n):
    nt = (n // 16) // 8 * 8
    rem = n - nt * 16
    pltpu.sync_copy(sh.at[pl.ds(s * nt, nt)], out_hbm.at[c, pl.ds(s * nt, nt)])
    if rem:
        @pl.when(s == 15)
        def _():
            pltpu.sync_copy(sh.at[pl.ds(nt * 16, rem)],
                            out_hbm.at[c, pl.ds(nt * 16, rem)])


def _scatter_pipeline(w, nch, idx_s, idx_d, bufs, sh, gsems, ssems,
                      load_chunk, src_hbm, dst_hbm):
    r0 = w * nch

    def load_idx(q, half):
        if src_hbm is not None:
            pltpu.sync_copy(src_hbm.at[pl.ds(r0 + 4 * q, 4)],
                            idx_s.at[pl.ds(4 * half, 4)])
        pltpu.sync_copy(dst_hbm.at[pl.ds(r0 + 4 * q, 4)],
                        idx_d.at[pl.ds(4 * half, 4)])

    def scat(t, half):
        return pltpu.async_copy(bufs[t], sh.at[idx_d.at[4 * half + t]],
                                ssems[t], add=True)

    def wait_scat(t, half):
        pltpu.make_async_copy(bufs[t], sh.at[idx_d.at[4 * half + t]],
                              ssems[t]).wait()

    def quadstep(q, half, first):
        load_idx(q, half)
        dgs = []
        for t in range(4):
            if not first:
                wait_scat(t, 1 - half)
            dgs.append(load_chunk(4 * q + t, 4 * half + t, bufs[t], gsems[t]))
        for t in range(4):
            dgs[t].wait()
            scat(t, half)

    nquad = nch // 4
    @pl.loop(0, nquad // 2)
    def _(m):
        @pl.when(m == 0)
        def _():
            quadstep(0, 0, True)

        @pl.when(m > 0)
        def _():
            quadstep(2 * m, 0, False)
        quadstep(2 * m + 1, 1, False)

    rem_q = nquad // 2 * 2
    for q in range(rem_q, nquad):
        quadstep(q, 0, False)
    last_half = 0 if rem_q < nquad else 1
    for t in range(4):
        wait_scat(t, last_half)
    for k in range(nquad * 4, nch):
        if src_hbm is not None:
            pltpu.sync_copy(src_hbm.at[pl.ds(r0 + k, 1)], idx_s.at[pl.ds(0, 1)])
        pltpu.sync_copy(dst_hbm.at[pl.ds(r0 + k, 1)], idx_d.at[pl.ds(0, 1)])
        load_chunk(k, 0, bufs[0], gsems[0]).wait()
        scat(0, 0).wait()


def _sc_srows_body(ept, n, x_hbm, src_hbm, dst_hbm, sp_hbm,
                   idx_s, idx_d, rows0, rows1, rows2, rows3, s_sh,
                   gs0, gs1, gs2, gs3, ss0, ss1, ss2, ss3):
    c, s, w = _tile_ids()
    nch = ept // _IW

    _zero_rows(rows0, _IW, 8)
    _zero_shared(rows0, s_sh, s, n)
    plsc.subcore_barrier()

    def load_chunk(k, idx_row, buf, sem):
        return pltpu.async_copy(x_hbm.at[idx_s.at[idx_row]], buf, sem)

    _scatter_pipeline(w, nch, idx_s, idx_d, (rows0, rows1, rows2, rows3),
                      s_sh, (gs0, gs1, gs2, gs3), (ss0, ss1, ss2, ss3),
                      load_chunk, src_hbm, dst_hbm)

    plsc.subcore_barrier()
    _write_shared(s_sh, sp_hbm, c, s, n)


def _sc_gpack_body(ept, n, g_hbm, dst_hbm, gp_hbm,
                   idx_d, gbuf0, gbuf1, gbuf2, gbuf3, g_sh,
                   gl0, gl1, gl2, gl3, gs0, gs1, gs2, gs3):
    c, s, w = _tile_ids()
    nch = ept // _IW

    _zero_rows(gbuf0, _IW, 6)
    _zero_shared(gbuf0, g_sh, s, n)
    plsc.subcore_barrier()

    e0 = w * ept

    def load_chunk(k, idx_row, buf, sem):
        return pltpu.async_copy(
            g_hbm.at[pl.ds(e0 + k * _IW, _IW)], buf, sem)

    _scatter_pipeline(w, nch, None, idx_d, (gbuf0, gbuf1, gbuf2, gbuf3),
                      g_sh, (gl0, gl1, gl2, gl3), (gs0, gs1, gs2, gs3),
                      load_chunk, None, dst_hbm)

    plsc.subcore_barrier()
    _write_shared(g_sh, gp_hbm, c, s, n)


def _sc_pred_body(ept, a_hbm, b_hbm, p_hbm, src_hbm, dst_hbm, w2_hbm, out_hbm,
                  idx_s, idx_d, ab0, bb0, pb0, ab1, bb1, pb1, ab2, bb2, pb2,
                  obuf, tmp, w2v,
                  sa0, sb0, sp0, sa1, sb1, sp1, sa2, sb2, sp2):
    c, s, w = _tile_ids()
    nch = ept // _IW
    pltpu.sync_copy(w2_hbm, w2v)
    p2bv = w2v[pl.ds(128, 16)]
    lane = lax.iota(jnp.int32, 16)
    zlane = jnp.zeros((16,), jnp.int32)

    r0 = w * nch
    e0 = w * ept
    bufs = ((ab0, bb0, pb0, sa0, sb0, sp0),
            (ab1, bb1, pb1, sa1, sb1, sp1),
            (ab2, bb2, pb2, sa2, sb2, sp2))

    def start(k, j):
        ab, bb, pb, sema, semb, semp = bufs[j]
        da = pltpu.async_copy(a_hbm.at[idx_s.at[j]], ab, sema)
        db = pltpu.async_copy(b_hbm.at[idx_d.at[j]], bb, semb)
        dp = pltpu.async_copy(p_hbm.at[pl.ds(e0 + k * _IW, _IW)], pb, semp)
        return (da, db, dp)

    def compute(k, j):
        ab, bb, pb = bufs[j][:3]

        @pl.loop(0, _IW, step=2)
        def _(e):
            acc0 = jnp.zeros((16,), _F32)
            acc1 = jnp.zeros((16,), _F32)
            for jj in range(8):
                sl = pl.ds(jj * 16, 16)
                w2c = w2v[sl]
                u0 = ab[e, sl] + bb[e, sl] + pb[e, sl]
                u1 = ab[e + 1, sl] + bb[e + 1, sl] + pb[e + 1, sl]
                acc0 = acc0 + jnp.maximum(u0, 0.0) * w2c
                acc1 = acc1 + jnp.maximum(u1, 0.0) * w2c
            tmp[e, :] = jnp.full((16,), jnp.sum(acc0), _F32)
            tmp[e + 1, :] = jnp.full((16,), jnp.sum(acc1), _F32)

    def sigmoid_pass(k):
        @pl.loop(0, _IW // 16)
        def _(i):
            t = plsc.load_gather(tmp, [i * 16 + lane, zlane]) + p2bv
            obuf[pl.ds(k * _IW + i * 16, 16)] = 1.0 / (1.0 + jnp.exp(-t))

    ngroup = nch // 3
    pltpu.sync_copy(src_hbm.at[pl.ds(r0, 3)], idx_s)
    pltpu.sync_copy(dst_hbm.at[pl.ds(r0, 3)], idx_d)
    start(0, 0)

    def wait0(k0):
        ab, bb, pb, sema, semb, semp = bufs[0]
        pltpu.make_async_copy(a_hbm.at[idx_s.at[0]], ab, sema).wait()
        pltpu.make_async_copy(b_hbm.at[idx_d.at[0]], bb, semb).wait()
        pltpu.make_async_copy(p_hbm.at[pl.ds(e0 + k0 * _IW, _IW)], pb,
                              semp).wait()

    @pl.loop(0, ngroup)
    def _(i):
        k0 = 3 * i
        d1 = start(k0 + 1, 1)
        d2 = start(k0 + 2, 2)
        wait0(k0)
        compute(k0, 0)
        sigmoid_pass(k0)
        for d in d1:
            d.wait()
        compute(k0 + 1, 1)
        sigmoid_pass(k0 + 1)
        for d in d2:
            d.wait()

        @pl.when(i < ngroup - 1)
        def _():
            pltpu.sync_copy(src_hbm.at[pl.ds(r0 + k0 + 3, 3)], idx_s)
            pltpu.sync_copy(dst_hbm.at[pl.ds(r0 + k0 + 3, 3)], idx_d)
            start(k0 + 3, 0)
        compute(k0 + 2, 2)
        sigmoid_pass(k0 + 2)

    for k in range(nch // 3 * 3, nch):
        j = k - nch // 3 * 3
        pltpu.sync_copy(src_hbm.at[pl.ds(r0 + k, 1)], idx_s.at[pl.ds(j, 1)])
        pltpu.sync_copy(dst_hbm.at[pl.ds(r0 + k, 1)], idx_d.at[pl.ds(j, 1)])
        for d in start(k, j):
            d.wait()
        compute(k, j)
        sigmoid_pass(k)

    pltpu.sync_copy(obuf, out_hbm.at[pl.ds(e0, ept)])


def kernel(x, edge_index, edge_trust_score, edge_query_embedding,
           c1_nW, c1_nb, c1_e1W, c1_e1b, c1_e2W, c1_e2b,
           c2_nW, c2_nb, c2_e1W, c2_e1b, c2_e2W, c2_e2b,
           p1_W, p1_b, p2_W, p2_b):
    n, din = x.shape
    e = edge_index.shape[1]
    h_dim = c1_nW.shape[1]
    dout = c2_nW.shape[1]
    q = edge_query_embedding.shape[1]
    ept = e // _NW

    src = edge_index[0]
    dst = edge_index[1]
    src2 = src.reshape(e // _IW, _IW)
    dst2 = dst.reshape(e // _IW, _IW)

    def pad_e2(e2w, e2b):
        return jnp.concatenate(
            [e2w, e2b[None, :], jnp.zeros((13, e2w.shape[1]), _F32)], axis=0)

    c1_e2Wp = pad_e2(c1_e2W, c1_e2b)
    c2_e2Wp = pad_e2(c2_e2W, c2_e2b)

    be = 2000
    g_w = 2 * (c1_e1W.shape[1] + 14)
    edge_pre = pl.pallas_call(
        _edge_pre_body,
        grid=(e // be,),
        in_specs=[
            pl.BlockSpec((be, 1), lambda i: (i, 0)),
            pl.BlockSpec((be, q), lambda i: (i, 0)),
            pl.BlockSpec(c1_e1W.shape, lambda i: (0, 0)),
            pl.BlockSpec((1, 34), lambda i: (0, 0)),
            pl.BlockSpec(c2_e1W.shape, lambda i: (0, 0)),
            pl.BlockSpec((1, 34), lambda i: (0, 0)),
            pl.BlockSpec((q, h_dim), lambda i: (0, 0)),
            pl.BlockSpec((1, h_dim), lambda i: (0, 0)),
        ],
        out_specs=[
            pl.BlockSpec((be, g_w), lambda i: (i, 0)),
            pl.BlockSpec((be, h_dim), lambda i: (i, 0)),
        ],
        out_shape=[
            jax.ShapeDtypeStruct((e, g_w), _F32),
            jax.ShapeDtypeStruct((e, h_dim), _F32),
        ],
    )
    gx, p_tab = edge_pre(
        edge_trust_score, edge_query_embedding,
        c1_e1W, c1_e1b.reshape(1, -1), c2_e1W, c2_e1b.reshape(1, -1),
        p1_W[2 * dout:], p1_b.reshape(1, -1))

    mesh = plsc.VectorSubcoreMesh(core_axis_name="core",
                                  subcore_axis_name="subcore")
    sc_params = pltpu.CompilerParams(use_tc_tiling_on_sc=False,
                                     needs_layout_passes=False)
    nch = ept // _IW
    dma = pltpu.SemaphoreType.DMA

    sc_srows = pl.kernel(
        functools.partial(_sc_srows_body, ept, n),
        out_type=jax.ShapeDtypeStruct((2, n, din), _F32),
        mesh=mesh,
        scratch_types=(
            [pltpu.VMEM((8, _IW), jnp.int32)] * 2
            + [pltpu.VMEM((_IW, din), _F32)] * 4
            + [pltpu.VMEM_SHARED((n, din), _F32)]
            + [dma] * 8
        ),
        compiler_params=sc_params,
    )

    sc_gpack = pl.kernel(
        functools.partial(_sc_gpack_body, ept, n),
        out_type=jax.ShapeDtypeStruct((2, n, g_w), _F32),
        mesh=mesh,
        scratch_types=(
            [pltpu.VMEM((8, _IW), jnp.int32)]
            + [pltpu.VMEM((_IW, g_w), _F32)] * 4
            + [pltpu.VMEM_SHARED((n, g_w), _F32)]
            + [dma] * 8
        ),
        compiler_params=sc_params,
    )

    bn = 2000
    conv_update = pl.pallas_call(
        _conv_update_body,
        grid=(n // bn,),
        in_specs=[
            pl.BlockSpec((2, bn, din), lambda i: (0, i, 0)),
            pl.BlockSpec((2, bn, g_w // 2), lambda i: (0, i, 0)),
            pl.BlockSpec((g_w // 2, din), lambda i: (0, 0)),
            pl.BlockSpec((din, h_dim), lambda i: (0, 0)),
            pl.BlockSpec((1, h_dim), lambda i: (0, 0)),
        ],
        out_specs=pl.BlockSpec((bn, h_dim), lambda i: (i, 0)),
        out_shape=jax.ShapeDtypeStruct((n, h_dim), _F32),
    )
    conv_ab = pl.pallas_call(
        _conv_ab_body,
        grid=(n // bn,),
        in_specs=[
            pl.BlockSpec((2, bn, h_dim), lambda i: (0, i, 0)),
            pl.BlockSpec((2, bn, g_w // 2), lambda i: (0, i, 0)),
            pl.BlockSpec((g_w // 2, h_dim), lambda i: (0, 0)),
            pl.BlockSpec((h_dim, dout), lambda i: (0, 0)),
            pl.BlockSpec((1, dout), lambda i: (0, 0)),
            pl.BlockSpec((dout, h_dim), lambda i: (0, 0)),
            pl.BlockSpec((dout, h_dim), lambda i: (0, 0)),
        ],
        out_specs=[
            pl.BlockSpec((bn, h_dim), lambda i: (i, 0)),
            pl.BlockSpec((bn, h_dim), lambda i: (i, 0)),
        ],
        out_shape=[
            jax.ShapeDtypeStruct((n, h_dim), _F32),
            jax.ShapeDtypeStruct((n, h_dim), _F32),
        ],
    )

    sp1 = sc_srows(x, src2, dst2)
    gp = sc_gpack(gx, dst2)
    gp1 = gp[:, :, :g_w // 2]
    gp2 = gp[:, :, g_w // 2:]
    h = conv_update(sp1, gp1, c1_e2Wp, c1_nW, c1_nb.reshape(1, -1))
    sp2 = sc_srows(h, src2, dst2)
    a_tab, b_tab = conv_ab(sp2, gp2, c2_e2Wp, c2_nW, c2_nb.reshape(1, -1),
                           p1_W[:dout], p1_W[dout:2 * dout])

    w2ext = jnp.concatenate(
        [p2_W[:, 0], jnp.full((16,), p2_b[0], _F32)])
    sc_pred = pl.kernel(
        functools.partial(_sc_pred_body, ept),
        out_type=jax.ShapeDtypeStruct((e,), _F32),
        mesh=mesh,
        scratch_types=(
            [pltpu.VMEM((3, _IW), jnp.int32)] * 2
            + [pltpu.VMEM((_IW, h_dim), _F32)] * 9
            + [pltpu.VMEM((ept,), _F32), pltpu.VMEM((_IW, 16), _F32),
               pltpu.VMEM((144,), _F32)]
            + [dma] * 9
        ),
        compiler_params=sc_params,
    )
    out = sc_pred(a_tab, b_tab, p_tab, src2, dst2, w2ext)
    return out.reshape(e, 1)
